# baseline (speedup 1.0000x reference)
import functools
import jax
import jax.numpy as jnp
from jax.experimental import pallas as pl

N = 10000
E = 160000
F_IN = 128
H = 32
G = 312

NB = 1000
EB = 640


def _bf16r(a):
    return a.astype(jnp.bfloat16).astype(jnp.float32)


def _dot(a, b):
    return jax.lax.dot_general(_bf16r(a), _bf16r(b), (((1,), (0,)), ((), ())),
                               preferred_element_type=jnp.float32,
                               precision=jax.lax.Precision.HIGHEST)


def _dot_exact(a, b):
    return jax.lax.dot_general(a, b, (((1,), (0,)), ((), ())),
                               preferred_element_type=jnp.float32,
                               precision=jax.lax.Precision.HIGHEST)


def _elu(x):
    return jnp.where(x > 0, x, jnp.exp(jnp.minimum(x, 0.0)) - 1.0)


def _tc1_body(x_ref, w0_ref, b0_ref, mw1_ref, mb1_ref, h_ref, t_ref):
    h = _dot(x_ref[...], w0_ref[...]) + b0_ref[...]
    h_ref[...] = h
    t_ref[...] = jnp.maximum(_dot(h, mw1_ref[...]) + mb1_ref[...], 0.0)


def _tc1(x, w0, b0, mw1, mb1):
    grid = (N // NB,)
    full = lambda a: pl.BlockSpec(a.shape, lambda i: (0,) * a.ndim)
    return pl.pallas_call(
        _tc1_body,
        grid=grid,
        in_specs=[pl.BlockSpec((NB, F_IN), lambda i: (i, 0)),
                  full(w0), full(b0), full(mw1), full(mb1)],
        out_specs=[pl.BlockSpec((NB, H), lambda i: (i, 0)),
                   pl.BlockSpec((NB, H), lambda i: (i, 0))],
        out_shape=[jax.ShapeDtypeStruct((N, H), jnp.float32),
                   jax.ShapeDtypeStruct((N, H), jnp.float32)],
    )(x, w0, b0, mw1, mb1)


def _tc2_body(t_ref, agg_ref, h_ref, mw2_ref, mb2_ref, xm_ref, m_ref):
    m = jax.nn.sigmoid(_dot(t_ref[...] + agg_ref[...], mw2_ref[...]) + mb2_ref[...])
    m_ref[...] = m
    xm_ref[...] = h_ref[...] * m


def _tc2(t, agg, h, mw2, mb2):
    grid = (N // NB,)
    full = lambda a: pl.BlockSpec(a.shape, lambda i: (0,) * a.ndim)
    return pl.pallas_call(
        _tc2_body,
        grid=grid,
        in_specs=[pl.BlockSpec((NB, H), lambda i: (i, 0)),
                  pl.BlockSpec((NB, H), lambda i: (i, 0)),
                  pl.BlockSpec((NB, H), lambda i: (i, 0)),
                  full(mw2), full(mb2)],
        out_specs=[pl.BlockSpec((NB, H), lambda i: (i, 0)),
                   pl.BlockSpec((NB, 1), lambda i: (i, 0))],
        out_shape=[jax.ShapeDtypeStruct((N, H), jnp.float32),
                   jax.ShapeDtypeStruct((N, 1), jnp.float32)],
    )(t, agg, h, mw2, mb2)


def _tc3_body(ea_ref, xs_ref, a1_ref, c1_ref, a2p_ref, c2p_ref, sel_ref, msg_ref):
    u = jnp.maximum(_dot(ea_ref[...], a1_ref[...]) + c1_ref[...], 0.0)
    w = _dot(u, a2p_ref[...]) + c2p_ref[...]
    xs = xs_ref[...]
    xt = jnp.concatenate([xs] * H, axis=1)
    msg_ref[...] = _dot_exact(_bf16r(w) * _bf16r(xt), sel_ref[...])


def _tc3(ea, xs, a1, c1, a2p, c2p, sel):
    grid = (E // EB,)
    full = lambda a: pl.BlockSpec(a.shape, lambda i: (0,) * a.ndim)
    return pl.pallas_call(
        _tc3_body,
        grid=grid,
        in_specs=[pl.BlockSpec((EB, 5), lambda i: (i, 0)),
                  pl.BlockSpec((EB, H), lambda i: (i, 0)),
                  full(a1), full(c1), full(a2p), full(c2p), full(sel)],
        out_specs=pl.BlockSpec((EB, H), lambda i: (i, 0)),
        out_shape=jax.ShapeDtypeStruct((E, H), jnp.float32),
    )(ea, xs, a1, c1, a2p, c2p, sel)


def _tc4_body(xm_ref, agg_ref, wroot_ref, m_ref, mw1_ref, mb1_ref, h_ref, t_ref):
    h = _elu(_dot(xm_ref[...], wroot_ref[...]) + agg_ref[...])
    h_ref[...] = h
    hm = h * m_ref[...]
    t_ref[...] = jnp.maximum(_dot(hm, mw1_ref[...]) + mb1_ref[...], 0.0)


def _tc4(xm, agg, wroot, m, mw1, mb1):
    grid = (N // NB,)
    full = lambda a: pl.BlockSpec(a.shape, lambda i: (0,) * a.ndim)
    return pl.pallas_call(
        _tc4_body,
        grid=grid,
        in_specs=[pl.BlockSpec((NB, H), lambda i: (i, 0)),
                  pl.BlockSpec((NB, H), lambda i: (i, 0)),
                  full(wroot),
                  pl.BlockSpec((NB, 1), lambda i: (i, 0)),
                  full(mw1), full(mb1)],
        out_specs=[pl.BlockSpec((NB, H), lambda i: (i, 0)),
                   pl.BlockSpec((NB, H), lambda i: (i, 0))],
        out_shape=[jax.ShapeDtypeStruct((N, H), jnp.float32),
                   jax.ShapeDtypeStruct((N, H), jnp.float32)],
    )(xm, agg, wroot, m, mw1, mb1)


def _tc4f_body(xm_ref, agg_ref, wroot_ref, h_ref):
    h_ref[...] = _elu(_dot(xm_ref[...], wroot_ref[...]) + agg_ref[...])


def _tc4f(xm, agg, wroot):
    grid = (N // NB,)
    full = lambda a: pl.BlockSpec(a.shape, lambda i: (0,) * a.ndim)
    return pl.pallas_call(
        _tc4f_body,
        grid=grid,
        in_specs=[pl.BlockSpec((NB, H), lambda i: (i, 0)),
                  pl.BlockSpec((NB, H), lambda i: (i, 0)),
                  full(wroot)],
        out_specs=pl.BlockSpec((NB, H), lambda i: (i, 0)),
        out_shape=jax.ShapeDtypeStruct((N, H), jnp.float32),
    )(xm, agg, wroot)


def _tc5_body(p_ref, w1_ref, b1_ref, w2_ref, b2_ref, w3_ref, b3_ref, o_ref):
    o = _elu(_dot(p_ref[...], w1_ref[...]) + b1_ref[...])
    o = _elu(_dot(o, w2_ref[...]) + b2_ref[...])
    o_ref[...] = _dot(o, w3_ref[...]) + b3_ref[...]


def _tc5(pooled, w1, b1, w2, b2, w3, b3):
    full = lambda a: pl.BlockSpec(a.shape, lambda *_: (0,) * a.ndim)
    return pl.pallas_call(
        _tc5_body,
        in_specs=[full(pooled), full(w1), full(b1), full(w2), full(b2),
                  full(w3), full(b3)],
        out_specs=full(jnp.zeros((G, 1))),
        out_shape=jax.ShapeDtypeStruct((G, 1), jnp.float32),
    )(pooled, w1, b1, w2, b2, w3, b3)


def _segsum_gather(table, src, dst, nseg):
    return jax.ops.segment_sum(table[src], dst, num_segments=nseg)


def _gather(table, src):
    return table[src]


def _segsum(vals, dst, nseg):
    return jax.ops.segment_sum(vals, dst, num_segments=nseg)


def kernel(x, edge_index, edge_attr, batch, params):
    src = edge_index[0]
    dst = edge_index[1]
    p = params
    row = lambda v: v.reshape(1, -1)

    a2p = [p["A2"][i].reshape(F_IN, H, H).transpose(0, 2, 1).reshape(F_IN, H * H)
           for i in range(3)]
    c2p = [p["c2"][i].reshape(H, H).T.reshape(1, H * H) for i in range(3)]
    sel = (jnp.arange(H * H, dtype=jnp.int32)[:, None] // H
           == jnp.arange(H, dtype=jnp.int32)[None, :]).astype(jnp.float32)

    h, t = _tc1(x, p["W0"], row(p["b0"]), p["Mw1"][0], row(p["Mb1"][0]))
    for i in range(3):
        agg = _segsum_gather(t, src, dst, N)
        xm, m = _tc2(t, agg, h, p["Mw2"][i], row(p["Mb2"][i]))
        xs = _gather(xm, src)
        msg = _tc3(edge_attr, xs, p["A1"][i], row(p["c1"][i]), a2p[i], c2p[i], sel)
        agg2 = _segsum(msg, dst, N)
        if i < 2:
            h, t = _tc4(xm, agg2, p["Wroot"][i], m,
                        p["Mw1"][i + 1], row(p["Mb1"][i + 1]))
        else:
            h = _tc4f(xm, agg2, p["Wroot"][i])
    pooled = _segsum(h, batch, G)
    o = _tc5(pooled, p["W1"], row(p["b1"]), p["W2"], row(p["b2"]),
             p["W3"], row(p["b3"]))
    return o.reshape(-1)


# speedup vs baseline: 1.5284x; 1.5284x over previous
import functools
import jax
import jax.numpy as jnp
from jax import lax
from jax.experimental import pallas as pl
from jax.experimental.pallas import tpu as pltpu
from jax.experimental.pallas import tpu_sc as plsc

N = 10000
E = 160000
F_IN = 128
H = 32
G = 312
W128 = 128

NB = 1000
EB = 640

NC = 2
NS = 16
NW = NC * NS
EPW = E // NW
ECH = 128
NCH = EPW // ECH
TAIL = EPW - NCH * ECH
NPAD = 10112
NROW = NPAD // NS
GPAD = 384
GROW = GPAD // NS
PCH = 40
PNCH = N // PCH
PITER = (PNCH + NW - 1) // NW


def _bf16r(a):
    return a.astype(jnp.bfloat16).astype(jnp.float32)


def _dot(a, b):
    return jax.lax.dot_general(_bf16r(a), _bf16r(b), (((1,), (0,)), ((), ())),
                               preferred_element_type=jnp.float32,
                               precision=jax.lax.Precision.HIGHEST)


def _dot_exact(a, b):
    return jax.lax.dot_general(a, b, (((1,), (0,)), ((), ())),
                               preferred_element_type=jnp.float32,
                               precision=jax.lax.Precision.HIGHEST)


def _elu(x):
    return jnp.where(x > 0, x, jnp.exp(jnp.minimum(x, 0.0)) - 1.0)


def _pad_lanes(a, nb):
    return jnp.concatenate([a, jnp.zeros((nb, W128 - H), jnp.float32)], axis=1)


def _tc1_body(x_ref, w0_ref, b0_ref, mw1_ref, mb1_ref, h_ref, t_ref):
    h = _dot(x_ref[...], w0_ref[...]) + b0_ref[...]
    h_ref[...] = h
    t_ref[...] = jnp.maximum(_dot(h, mw1_ref[...]) + mb1_ref[...], 0.0)


def _tc1(x, w0, b0, mw1p, mb1p):
    grid = (N // NB,)
    full = lambda a: pl.BlockSpec(a.shape, lambda i: (0,) * a.ndim)
    return pl.pallas_call(
        _tc1_body,
        grid=grid,
        in_specs=[pl.BlockSpec((NB, F_IN), lambda i: (i, 0)),
                  full(w0), full(b0), full(mw1p), full(mb1p)],
        out_specs=[pl.BlockSpec((NB, H), lambda i: (i, 0)),
                   pl.BlockSpec((NB, W128), lambda i: (i, 0))],
        out_shape=[jax.ShapeDtypeStruct((N, H), jnp.float32),
                   jax.ShapeDtypeStruct((N, W128), jnp.float32)],
    )(x, w0, b0, mw1p, mb1p)


def _tc2_body(t_ref, agg_ref, h_ref, mw2_ref, mb2_ref, xm_ref, m_ref):
    aggs = agg_ref[...]
    agg = (aggs[0] + aggs[1])[:, :H]
    t = t_ref[...][:, :H]
    m = jax.nn.sigmoid(_dot(t + agg, mw2_ref[...]) + mb2_ref[...])
    m_ref[...] = m
    xm_ref[...] = _pad_lanes(h_ref[...] * m, NB)


def _tc2(t, agg, h, mw2, mb2):
    grid = (N // NB,)
    full = lambda a: pl.BlockSpec(a.shape, lambda i: (0,) * a.ndim)
    return pl.pallas_call(
        _tc2_body,
        grid=grid,
        in_specs=[pl.BlockSpec((NB, W128), lambda i: (i, 0)),
                  pl.BlockSpec((NC, NB, W128), lambda i: (0, i, 0)),
                  pl.BlockSpec((NB, H), lambda i: (i, 0)),
                  full(mw2), full(mb2)],
        out_specs=[pl.BlockSpec((NB, W128), lambda i: (i, 0)),
                   pl.BlockSpec((NB, 1), lambda i: (i, 0))],
        out_shape=[jax.ShapeDtypeStruct((N, W128), jnp.float32),
                   jax.ShapeDtypeStruct((N, 1), jnp.float32)],
    )(t, agg, h, mw2, mb2)


def _tc3_body(ea_ref, xs_ref, a1_ref, c1_ref, a2p_ref, c2p_ref, sel_ref, msg_ref):
    u = jnp.maximum(_dot(ea_ref[...], a1_ref[...]) + c1_ref[...], 0.0)
    w = _dot(u, a2p_ref[...]) + c2p_ref[...]
    xs = xs_ref[...][:, :H]
    xt = jnp.concatenate([xs] * H, axis=1)
    msg = _dot_exact(_bf16r(w) * _bf16r(xt), sel_ref[...])
    msg_ref[...] = _pad_lanes(msg, EB)


def _tc3(ea, xs, a1, c1, a2p, c2p, sel):
    grid = (E // EB,)
    full = lambda a: pl.BlockSpec(a.shape, lambda i: (0,) * a.ndim)
    return pl.pallas_call(
        _tc3_body,
        grid=grid,
        in_specs=[pl.BlockSpec((EB, 5), lambda i: (i, 0)),
                  pl.BlockSpec((EB, W128), lambda i: (i, 0)),
                  full(a1), full(c1), full(a2p), full(c2p), full(sel)],
        out_specs=pl.BlockSpec((EB, W128), lambda i: (i, 0)),
        out_shape=jax.ShapeDtypeStruct((E, W128), jnp.float32),
    )(ea, xs, a1, c1, a2p, c2p, sel)


def _tc4_body(xm_ref, agg_ref, wroot_ref, m_ref, mw1_ref, mb1_ref, h_ref, t_ref):
    aggs = agg_ref[...]
    agg = (aggs[0] + aggs[1])[:, :H]
    h = _elu(_dot(xm_ref[...][:, :H], wroot_ref[...]) + agg)
    h_ref[...] = h
    hm = h * m_ref[...]
    t_ref[...] = jnp.maximum(_dot(hm, mw1_ref[...]) + mb1_ref[...], 0.0)


def _tc4(xm, agg, wroot, m, mw1p, mb1p):
    grid = (N // NB,)
    full = lambda a: pl.BlockSpec(a.shape, lambda i: (0,) * a.ndim)
    return pl.pallas_call(
        _tc4_body,
        grid=grid,
        in_specs=[pl.BlockSpec((NB, W128), lambda i: (i, 0)),
                  pl.BlockSpec((NC, NB, W128), lambda i: (0, i, 0)),
                  full(wroot),
                  pl.BlockSpec((NB, 1), lambda i: (i, 0)),
                  full(mw1p), full(mb1p)],
        out_specs=[pl.BlockSpec((NB, H), lambda i: (i, 0)),
                   pl.BlockSpec((NB, W128), lambda i: (i, 0))],
        out_shape=[jax.ShapeDtypeStruct((N, H), jnp.float32),
                   jax.ShapeDtypeStruct((N, W128), jnp.float32)],
    )(xm, agg, wroot, m, mw1p, mb1p)


def _tc4f_body(xm_ref, agg_ref, wroot_ref, h_ref):
    aggs = agg_ref[...]
    agg = (aggs[0] + aggs[1])[:, :H]
    h_ref[...] = _pad_lanes(_elu(_dot(xm_ref[...][:, :H], wroot_ref[...]) + agg), NB)


def _tc4f(xm, agg, wroot):
    grid = (N // NB,)
    full = lambda a: pl.BlockSpec(a.shape, lambda i: (0,) * a.ndim)
    return pl.pallas_call(
        _tc4f_body,
        grid=grid,
        in_specs=[pl.BlockSpec((NB, W128), lambda i: (i, 0)),
                  pl.BlockSpec((NC, NB, W128), lambda i: (0, i, 0)),
                  full(wroot)],
        out_specs=pl.BlockSpec((NB, W128), lambda i: (i, 0)),
        out_shape=jax.ShapeDtypeStruct((N, W128), jnp.float32),
    )(xm, agg, wroot)


def _tc5_body(p_ref, w1_ref, b1_ref, w2_ref, b2_ref, w3_ref, b3_ref, o_ref):
    ps = p_ref[...]
    pooled = (ps[0] + ps[1])[:G, :H]
    o = _elu(_dot(pooled, w1_ref[...]) + b1_ref[...])
    o = _elu(_dot(o, w2_ref[...]) + b2_ref[...])
    o_ref[...] = _dot(o, w3_ref[...]) + b3_ref[...]


def _tc5(pooled, w1, b1, w2, b2, w3, b3):
    full = lambda a: pl.BlockSpec(a.shape, lambda *_: (0,) * a.ndim)
    return pl.pallas_call(
        _tc5_body,
        in_specs=[full(pooled), full(w1), full(b1), full(w2), full(b2),
                  full(w3), full(b3)],
        out_specs=full(jnp.zeros((G, 1))),
        out_shape=jax.ShapeDtypeStruct((G, 1), jnp.float32),
    )(pooled, w1, b1, w2, b2, w3, b3)


def _sc_mesh():
    return plsc.VectorSubcoreMesh(core_axis_name="c", subcore_axis_name="s")


@functools.partial(
    pl.kernel,
    out_type=jax.ShapeDtypeStruct((NC, NPAD, W128), jnp.float32),
    mesh=_sc_mesh(),
    scratch_types=[
        pltpu.VMEM((ECH,), jnp.int32),
        pltpu.VMEM((ECH,), jnp.int32),
        pltpu.VMEM((ECH, W128), jnp.float32),
        pltpu.VMEM((TAIL,), jnp.int32),
        pltpu.VMEM((TAIL,), jnp.int32),
        pltpu.VMEM((TAIL, W128), jnp.float32),
        pltpu.VMEM_SHARED((NPAD, W128), jnp.float32),
        pltpu.SemaphoreType.DMA,
    ],
)
def _sc_gather_segsum(t_hbm, src_hbm, dst_hbm, zeros_hbm, out_hbm,
                      gidx, didx, rows, gidx8, didx8, rows8, acc, sem):
    cid = lax.axis_index("c")
    sid = lax.axis_index("s")
    wid = cid * NS + sid
    rbase = sid * NROW

    pltpu.sync_copy(zeros_hbm.at[pl.ds(rbase, NROW)],
                    acc.at[pl.ds(rbase, NROW)])
    plsc.subcore_barrier()
    ebase = wid * EPW

    def body(j, carry):
        base = ebase + j * ECH
        pltpu.sync_copy(src_hbm.at[pl.ds(base, ECH)], gidx)
        pltpu.sync_copy(dst_hbm.at[pl.ds(base, ECH)], didx)
        pltpu.async_copy(t_hbm.at[gidx], rows, sem).wait()
        pltpu.sync_copy(rows, acc.at[didx], add=True)
        return carry

    lax.fori_loop(0, NCH, body, 0)
    base = ebase + NCH * ECH
    pltpu.sync_copy(src_hbm.at[pl.ds(base, TAIL)], gidx8)
    pltpu.sync_copy(dst_hbm.at[pl.ds(base, TAIL)], didx8)
    pltpu.async_copy(t_hbm.at[gidx8], rows8, sem).wait()
    pltpu.sync_copy(rows8, acc.at[didx8], add=True)
    plsc.subcore_barrier()
    pltpu.sync_copy(acc.at[pl.ds(rbase, NROW)],
                    out_hbm.at[cid].at[pl.ds(rbase, NROW)])


@functools.partial(
    pl.kernel,
    out_type=jax.ShapeDtypeStruct((E, W128), jnp.float32),
    mesh=_sc_mesh(),
    scratch_types=[
        pltpu.VMEM((ECH,), jnp.int32),
        pltpu.VMEM((ECH, W128), jnp.float32),
        pltpu.VMEM((TAIL,), jnp.int32),
        pltpu.VMEM((TAIL, W128), jnp.float32),
        pltpu.SemaphoreType.DMA,
    ],
)
def _sc_gather(t_hbm, src_hbm, out_hbm, gidx, rows, gidx8, rows8, sem):
    cid = lax.axis_index("c")
    sid = lax.axis_index("s")
    wid = cid * NS + sid
    ebase = wid * EPW

    def body(j, carry):
        base = ebase + j * ECH
        pltpu.sync_copy(src_hbm.at[pl.ds(base, ECH)], gidx)
        pltpu.async_copy(t_hbm.at[gidx], rows, sem).wait()
        pltpu.sync_copy(rows, out_hbm.at[pl.ds(base, ECH)])
        return carry

    lax.fori_loop(0, NCH, body, 0)
    base = ebase + NCH * ECH
    pltpu.sync_copy(src_hbm.at[pl.ds(base, TAIL)], gidx8)
    pltpu.async_copy(t_hbm.at[gidx8], rows8, sem).wait()
    pltpu.sync_copy(rows8, out_hbm.at[pl.ds(base, TAIL)])


@functools.partial(
    pl.kernel,
    out_type=jax.ShapeDtypeStruct((NC, NPAD, W128), jnp.float32),
    mesh=_sc_mesh(),
    scratch_types=[
        pltpu.VMEM((ECH,), jnp.int32),
        pltpu.VMEM((ECH, W128), jnp.float32),
        pltpu.VMEM((TAIL,), jnp.int32),
        pltpu.VMEM((TAIL, W128), jnp.float32),
        pltpu.VMEM_SHARED((NPAD, W128), jnp.float32),
    ],
)
def _sc_segsum(vals_hbm, dst_hbm, zeros_hbm, out_hbm,
               didx, rows, didx8, rows8, acc):
    cid = lax.axis_index("c")
    sid = lax.axis_index("s")
    wid = cid * NS + sid
    rbase = sid * NROW

    pltpu.sync_copy(zeros_hbm.at[pl.ds(rbase, NROW)],
                    acc.at[pl.ds(rbase, NROW)])
    plsc.subcore_barrier()
    ebase = wid * EPW

    def body(j, carry):
        base = ebase + j * ECH
        pltpu.sync_copy(dst_hbm.at[pl.ds(base, ECH)], didx)
        pltpu.sync_copy(vals_hbm.at[pl.ds(base, ECH)], rows)
        pltpu.sync_copy(rows, acc.at[didx], add=True)
        return carry

    lax.fori_loop(0, NCH, body, 0)
    base = ebase + NCH * ECH
    pltpu.sync_copy(dst_hbm.at[pl.ds(base, TAIL)], didx8)
    pltpu.sync_copy(vals_hbm.at[pl.ds(base, TAIL)], rows8)
    pltpu.sync_copy(rows8, acc.at[didx8], add=True)
    plsc.subcore_barrier()
    pltpu.sync_copy(acc.at[pl.ds(rbase, NROW)],
                    out_hbm.at[cid].at[pl.ds(rbase, NROW)])


@functools.partial(
    pl.kernel,
    out_type=jax.ShapeDtypeStruct((NC, GPAD, W128), jnp.float32),
    mesh=_sc_mesh(),
    scratch_types=[
        pltpu.VMEM((PCH,), jnp.int32),
        pltpu.VMEM((PCH, W128), jnp.float32),
        pltpu.VMEM_SHARED((GPAD, W128), jnp.float32),
    ],
)
def _sc_pool(h_hbm, batch_hbm, zeros_hbm, out_hbm, bidx, rows, acc):
    cid = lax.axis_index("c")
    sid = lax.axis_index("s")
    wid = cid * NS + sid
    rbase = sid * GROW

    pltpu.sync_copy(zeros_hbm.at[pl.ds(rbase, GROW)],
                    acc.at[pl.ds(rbase, GROW)])
    plsc.subcore_barrier()

    def body(k, carry):
        c = wid + k * NW

        @pl.when(c < PNCH)
        def _():
            base = c * PCH
            pltpu.sync_copy(batch_hbm.at[pl.ds(base, PCH)], bidx)
            pltpu.sync_copy(h_hbm.at[pl.ds(base, PCH)], rows)
            pltpu.sync_copy(rows, acc.at[bidx], add=True)

        return carry

    lax.fori_loop(0, PITER, body, 0)
    plsc.subcore_barrier()
    pltpu.sync_copy(acc.at[pl.ds(rbase, GROW)],
                    out_hbm.at[cid].at[pl.ds(rbase, GROW)])


def kernel(x, edge_index, edge_attr, batch, params):
    src = edge_index[0]
    dst = edge_index[1]
    p = params
    row = lambda v: v.reshape(1, -1)

    a2p = [p["A2"][i].reshape(F_IN, H, H).transpose(0, 2, 1).reshape(F_IN, H * H)
           for i in range(3)]
    c2p = [p["c2"][i].reshape(H, H).T.reshape(1, H * H) for i in range(3)]
    sel = (jnp.arange(H * H, dtype=jnp.int32)[:, None] // H
           == jnp.arange(H, dtype=jnp.int32)[None, :]).astype(jnp.float32)
    mw1p = [jnp.pad(p["Mw1"][i], ((0, 0), (0, W128 - H))) for i in range(3)]
    mb1p = [jnp.pad(row(p["Mb1"][i]), ((0, 0), (0, W128 - H))) for i in range(3)]

    zeros_n = jnp.zeros((NPAD, W128), jnp.float32)
    zeros_g = jnp.zeros((GPAD, W128), jnp.float32)

    h, t = _tc1(x, p["W0"], row(p["b0"]), mw1p[0], mb1p[0])
    for i in range(3):
        agg = _sc_gather_segsum(t, src, dst, zeros_n)
        xm, m = _tc2(t, agg, h, p["Mw2"][i], row(p["Mb2"][i]))
        xs = _sc_gather(xm, src)
        msg = _tc3(edge_attr, xs, p["A1"][i], row(p["c1"][i]), a2p[i], c2p[i], sel)
        agg2 = _sc_segsum(msg, dst, zeros_n)
        if i < 2:
            h, t = _tc4(xm, agg2, p["Wroot"][i], m, mw1p[i + 1], mb1p[i + 1])
        else:
            h = _tc4f(xm, agg2, p["Wroot"][i])
    pooled = _sc_pool(h, batch, zeros_g)
    o = _tc5(pooled, p["W1"], row(p["b1"]), p["W2"], row(p["b2"]),
             p["W3"], row(p["b3"]))
    return o.reshape(-1)


# speedup vs baseline: 3.3437x; 2.1877x over previous
import functools
import jax
import jax.numpy as jnp
from jax import lax
from jax.experimental import pallas as pl
from jax.experimental.pallas import tpu as pltpu
from jax.experimental.pallas import tpu_sc as plsc

N = 10000
E = 160000
F_IN = 128
H = 32
G = 312
W128 = 128

NB = 1000
EB = 640

NC = 2
NS = 16
NW = NC * NS
EPW = E // NW
ECH = 128
NCH = EPW // ECH
TAIL = EPW - NCH * ECH
NPAD = 10112
NROW = NPAD // NS
GPAD = 384
GROW = GPAD // NS
PCH = 40
PNCH = N // PCH
PITER = (PNCH + NW - 1) // NW


def _dot(a, b):
    return jax.lax.dot_general(a.astype(jnp.bfloat16), b.astype(jnp.bfloat16),
                               (((1,), (0,)), ((), ())),
                               preferred_element_type=jnp.float32)


def _elu(x):
    return jnp.where(x > 0, x, jnp.exp(jnp.minimum(x, 0.0)) - 1.0)


def _pad_lanes(a, nb):
    return jnp.concatenate([a, jnp.zeros((nb, W128 - H), jnp.float32)], axis=1)


def _tc1_body(x_ref, w0_ref, b0_ref, mw1_ref, mb1_ref, h_ref, t_ref):
    h = _dot(x_ref[...], w0_ref[...]) + b0_ref[...]
    h_ref[...] = h
    t_ref[...] = jnp.maximum(_dot(h, mw1_ref[...]) + mb1_ref[...], 0.0)


def _tc1(x, w0, b0, mw1p, mb1p):
    grid = (N // NB,)
    full = lambda a: pl.BlockSpec(a.shape, lambda i: (0,) * a.ndim)
    return pl.pallas_call(
        _tc1_body,
        grid=grid,
        in_specs=[pl.BlockSpec((NB, F_IN), lambda i: (i, 0)),
                  full(w0), full(b0), full(mw1p), full(mb1p)],
        out_specs=[pl.BlockSpec((NB, H), lambda i: (i, 0)),
                   pl.BlockSpec((NB, W128), lambda i: (i, 0))],
        out_shape=[jax.ShapeDtypeStruct((N, H), jnp.float32),
                   jax.ShapeDtypeStruct((N, W128), jnp.float32)],
    )(x, w0, b0, mw1p, mb1p)


def _tc2_body(t_ref, agg_ref, h_ref, mw2_ref, mb2_ref, xm_ref, m_ref):
    aggs = agg_ref[...]
    agg = (aggs[0] + aggs[1])[:, :H]
    t = t_ref[...][:, :H]
    m = jax.nn.sigmoid(_dot(t + agg, mw2_ref[...]) + mb2_ref[...])
    m_ref[...] = m
    xm_ref[...] = _pad_lanes(h_ref[...] * m, NB)


def _tc2(t, agg, h, mw2, mb2):
    grid = (N // NB,)
    full = lambda a: pl.BlockSpec(a.shape, lambda i: (0,) * a.ndim)
    return pl.pallas_call(
        _tc2_body,
        grid=grid,
        in_specs=[pl.BlockSpec((NB, W128), lambda i: (i, 0)),
                  pl.BlockSpec((NC, NB, W128), lambda i: (0, i, 0)),
                  pl.BlockSpec((NB, H), lambda i: (i, 0)),
                  full(mw2), full(mb2)],
        out_specs=[pl.BlockSpec((NB, W128), lambda i: (i, 0)),
                   pl.BlockSpec((NB, 1), lambda i: (i, 0))],
        out_shape=[jax.ShapeDtypeStruct((N, W128), jnp.float32),
                   jax.ShapeDtypeStruct((N, 1), jnp.float32)],
    )(t, agg, h, mw2, mb2)


def _tc3_body(ea_ref, xs_ref, a1_ref, c1_ref, a2p_ref, c2p_ref, sel_ref, msg_ref):
    u = jnp.maximum(_dot(ea_ref[...], a1_ref[...]) + c1_ref[...], 0.0)
    w = _dot(u, a2p_ref[...]) + c2p_ref[...]
    xs = xs_ref[...][:, :H]
    xt = jnp.concatenate([xs] * H, axis=1)
    prod = (w.astype(jnp.bfloat16).astype(jnp.float32)
            * xt.astype(jnp.bfloat16).astype(jnp.float32))
    ph = prod.astype(jnp.bfloat16)
    pr = (prod - ph.astype(jnp.float32)).astype(jnp.bfloat16)
    selb = sel_ref[...].astype(jnp.bfloat16)
    dims = (((1,), (0,)), ((), ()))
    msg = (jax.lax.dot_general(ph, selb, dims, preferred_element_type=jnp.float32)
           + jax.lax.dot_general(pr, selb, dims, preferred_element_type=jnp.float32))
    msg_ref[...] = _pad_lanes(msg, EB)


def _tc3(ea, xs, a1, c1, a2p, c2p, sel):
    grid = (E // EB,)
    full = lambda a: pl.BlockSpec(a.shape, lambda i: (0,) * a.ndim)
    return pl.pallas_call(
        _tc3_body,
        grid=grid,
        in_specs=[pl.BlockSpec((EB, 5), lambda i: (i, 0)),
                  pl.BlockSpec((EB, W128), lambda i: (i, 0)),
                  full(a1), full(c1), full(a2p), full(c2p), full(sel)],
        out_specs=pl.BlockSpec((EB, W128), lambda i: (i, 0)),
        out_shape=jax.ShapeDtypeStruct((E, W128), jnp.float32),
    )(ea, xs, a1, c1, a2p, c2p, sel)


def _tc4_body(xm_ref, agg_ref, wroot_ref, m_ref, mw1_ref, mb1_ref, h_ref, t_ref):
    aggs = agg_ref[...]
    agg = (aggs[0] + aggs[1])[:, :H]
    h = _elu(_dot(xm_ref[...][:, :H], wroot_ref[...]) + agg)
    h_ref[...] = h
    hm = h * m_ref[...]
    t_ref[...] = jnp.maximum(_dot(hm, mw1_ref[...]) + mb1_ref[...], 0.0)


def _tc4(xm, agg, wroot, m, mw1p, mb1p):
    grid = (N // NB,)
    full = lambda a: pl.BlockSpec(a.shape, lambda i: (0,) * a.ndim)
    return pl.pallas_call(
        _tc4_body,
        grid=grid,
        in_specs=[pl.BlockSpec((NB, W128), lambda i: (i, 0)),
                  pl.BlockSpec((NC, NB, W128), lambda i: (0, i, 0)),
                  full(wroot),
                  pl.BlockSpec((NB, 1), lambda i: (i, 0)),
                  full(mw1p), full(mb1p)],
        out_specs=[pl.BlockSpec((NB, H), lambda i: (i, 0)),
                   pl.BlockSpec((NB, W128), lambda i: (i, 0))],
        out_shape=[jax.ShapeDtypeStruct((N, H), jnp.float32),
                   jax.ShapeDtypeStruct((N, W128), jnp.float32)],
    )(xm, agg, wroot, m, mw1p, mb1p)


def _tc4f_body(xm_ref, agg_ref, wroot_ref, h_ref):
    aggs = agg_ref[...]
    agg = (aggs[0] + aggs[1])[:, :H]
    h_ref[...] = _pad_lanes(_elu(_dot(xm_ref[...][:, :H], wroot_ref[...]) + agg), NB)


def _tc4f(xm, agg, wroot):
    grid = (N // NB,)
    full = lambda a: pl.BlockSpec(a.shape, lambda i: (0,) * a.ndim)
    return pl.pallas_call(
        _tc4f_body,
        grid=grid,
        in_specs=[pl.BlockSpec((NB, W128), lambda i: (i, 0)),
                  pl.BlockSpec((NC, NB, W128), lambda i: (0, i, 0)),
                  full(wroot)],
        out_specs=pl.BlockSpec((NB, W128), lambda i: (i, 0)),
        out_shape=jax.ShapeDtypeStruct((N, W128), jnp.float32),
    )(xm, agg, wroot)


def _tc5_body(p_ref, w1_ref, b1_ref, w2_ref, b2_ref, w3_ref, b3_ref, o_ref):
    ps = p_ref[...]
    pooled = (ps[0] + ps[1])[:G, :H]
    o = _elu(_dot(pooled, w1_ref[...]) + b1_ref[...])
    o = _elu(_dot(o, w2_ref[...]) + b2_ref[...])
    o_ref[...] = _dot(o, w3_ref[...]) + b3_ref[...]


def _tc5(pooled, w1, b1, w2, b2, w3, b3):
    full = lambda a: pl.BlockSpec(a.shape, lambda *_: (0,) * a.ndim)
    return pl.pallas_call(
        _tc5_body,
        in_specs=[full(pooled), full(w1), full(b1), full(w2), full(b2),
                  full(w3), full(b3)],
        out_specs=full(jnp.zeros((G, 1))),
        out_shape=jax.ShapeDtypeStruct((G, 1), jnp.float32),
    )(pooled, w1, b1, w2, b2, w3, b3)


def _sc_mesh():
    return plsc.VectorSubcoreMesh(core_axis_name="c", subcore_axis_name="s")


@functools.partial(
    pl.kernel,
    out_type=jax.ShapeDtypeStruct((NC, NPAD, W128), jnp.float32),
    mesh=_sc_mesh(),
    scratch_types=[
        pltpu.VMEM((ECH,), jnp.int32),
        pltpu.VMEM((ECH,), jnp.int32),
        pltpu.VMEM((ECH, W128), jnp.float32),
        pltpu.VMEM((TAIL,), jnp.int32),
        pltpu.VMEM((TAIL,), jnp.int32),
        pltpu.VMEM((TAIL, W128), jnp.float32),
        pltpu.VMEM_SHARED((NPAD, W128), jnp.float32),
        pltpu.SemaphoreType.DMA,
    ],
)
def _sc_gather_segsum(t_hbm, src_hbm, dst_hbm, zeros_hbm, out_hbm,
                      gidx, didx, rows, gidx8, didx8, rows8, acc, sem):
    cid = lax.axis_index("c")
    sid = lax.axis_index("s")
    wid = cid * NS + sid
    rbase = sid * NROW

    pltpu.sync_copy(zeros_hbm.at[pl.ds(rbase, NROW)],
                    acc.at[pl.ds(rbase, NROW)])
    plsc.subcore_barrier()
    ebase = wid * EPW

    def body(j, carry):
        base = ebase + j * ECH
        pltpu.sync_copy(src_hbm.at[pl.ds(base, ECH)], gidx)
        pltpu.sync_copy(dst_hbm.at[pl.ds(base, ECH)], didx)
        pltpu.async_copy(t_hbm.at[gidx], rows, sem).wait()
        pltpu.sync_copy(rows, acc.at[didx], add=True)
        return carry

    lax.fori_loop(0, NCH, body, 0)
    base = ebase + NCH * ECH
    pltpu.sync_copy(src_hbm.at[pl.ds(base, TAIL)], gidx8)
    pltpu.sync_copy(dst_hbm.at[pl.ds(base, TAIL)], didx8)
    pltpu.async_copy(t_hbm.at[gidx8], rows8, sem).wait()
    pltpu.sync_copy(rows8, acc.at[didx8], add=True)
    plsc.subcore_barrier()
    pltpu.sync_copy(acc.at[pl.ds(rbase, NROW)],
                    out_hbm.at[cid].at[pl.ds(rbase, NROW)])


@functools.partial(
    pl.kernel,
    out_type=jax.ShapeDtypeStruct((E, W128), jnp.float32),
    mesh=_sc_mesh(),
    scratch_types=[
        pltpu.VMEM((ECH,), jnp.int32),
        pltpu.VMEM((ECH, W128), jnp.float32),
        pltpu.VMEM((TAIL,), jnp.int32),
        pltpu.VMEM((TAIL, W128), jnp.float32),
        pltpu.SemaphoreType.DMA,
    ],
)
def _sc_gather(t_hbm, src_hbm, out_hbm, gidx, rows, gidx8, rows8, sem):
    cid = lax.axis_index("c")
    sid = lax.axis_index("s")
    wid = cid * NS + sid
    ebase = wid * EPW

    def body(j, carry):
        base = ebase + j * ECH
        pltpu.sync_copy(src_hbm.at[pl.ds(base, ECH)], gidx)
        pltpu.async_copy(t_hbm.at[gidx], rows, sem).wait()
        pltpu.sync_copy(rows, out_hbm.at[pl.ds(base, ECH)])
        return carry

    lax.fori_loop(0, NCH, body, 0)
    base = ebase + NCH * ECH
    pltpu.sync_copy(src_hbm.at[pl.ds(base, TAIL)], gidx8)
    pltpu.async_copy(t_hbm.at[gidx8], rows8, sem).wait()
    pltpu.sync_copy(rows8, out_hbm.at[pl.ds(base, TAIL)])


@functools.partial(
    pl.kernel,
    out_type=jax.ShapeDtypeStruct((NC, NPAD, W128), jnp.float32),
    mesh=_sc_mesh(),
    scratch_types=[
        pltpu.VMEM((ECH,), jnp.int32),
        pltpu.VMEM((ECH, W128), jnp.float32),
        pltpu.VMEM((TAIL,), jnp.int32),
        pltpu.VMEM((TAIL, W128), jnp.float32),
        pltpu.VMEM_SHARED((NPAD, W128), jnp.float32),
    ],
)
def _sc_segsum(vals_hbm, dst_hbm, zeros_hbm, out_hbm,
               didx, rows, didx8, rows8, acc):
    cid = lax.axis_index("c")
    sid = lax.axis_index("s")
    wid = cid * NS + sid
    rbase = sid * NROW

    pltpu.sync_copy(zeros_hbm.at[pl.ds(rbase, NROW)],
                    acc.at[pl.ds(rbase, NROW)])
    plsc.subcore_barrier()
    ebase = wid * EPW

    def body(j, carry):
        base = ebase + j * ECH
        pltpu.sync_copy(dst_hbm.at[pl.ds(base, ECH)], didx)
        pltpu.sync_copy(vals_hbm.at[pl.ds(base, ECH)], rows)
        pltpu.sync_copy(rows, acc.at[didx], add=True)
        return carry

    lax.fori_loop(0, NCH, body, 0)
    base = ebase + NCH * ECH
    pltpu.sync_copy(dst_hbm.at[pl.ds(base, TAIL)], didx8)
    pltpu.sync_copy(vals_hbm.at[pl.ds(base, TAIL)], rows8)
    pltpu.sync_copy(rows8, acc.at[didx8], add=True)
    plsc.subcore_barrier()
    pltpu.sync_copy(acc.at[pl.ds(rbase, NROW)],
                    out_hbm.at[cid].at[pl.ds(rbase, NROW)])


@functools.partial(
    pl.kernel,
    out_type=jax.ShapeDtypeStruct((NC, GPAD, W128), jnp.float32),
    mesh=_sc_mesh(),
    scratch_types=[
        pltpu.VMEM((PCH,), jnp.int32),
        pltpu.VMEM((PCH, W128), jnp.float32),
        pltpu.VMEM_SHARED((GPAD, W128), jnp.float32),
    ],
)
def _sc_pool(h_hbm, batch_hbm, zeros_hbm, out_hbm, bidx, rows, acc):
    cid = lax.axis_index("c")
    sid = lax.axis_index("s")
    wid = cid * NS + sid
    rbase = sid * GROW

    pltpu.sync_copy(zeros_hbm.at[pl.ds(rbase, GROW)],
                    acc.at[pl.ds(rbase, GROW)])
    plsc.subcore_barrier()

    def body(k, carry):
        c = wid + k * NW

        @pl.when(c < PNCH)
        def _():
            base = c * PCH
            pltpu.sync_copy(batch_hbm.at[pl.ds(base, PCH)], bidx)
            pltpu.sync_copy(h_hbm.at[pl.ds(base, PCH)], rows)
            pltpu.sync_copy(rows, acc.at[bidx], add=True)

        return carry

    lax.fori_loop(0, PITER, body, 0)
    plsc.subcore_barrier()
    pltpu.sync_copy(acc.at[pl.ds(rbase, GROW)],
                    out_hbm.at[cid].at[pl.ds(rbase, GROW)])


def kernel(x, edge_index, edge_attr, batch, params):
    src = edge_index[0]
    dst = edge_index[1]
    p = params
    row = lambda v: v.reshape(1, -1)

    a2p = [p["A2"][i].reshape(F_IN, H, H).transpose(0, 2, 1).reshape(F_IN, H * H)
           for i in range(3)]
    c2p = [p["c2"][i].reshape(H, H).T.reshape(1, H * H) for i in range(3)]
    sel = (jnp.arange(H * H, dtype=jnp.int32)[:, None] // H
           == jnp.arange(H, dtype=jnp.int32)[None, :]).astype(jnp.float32)
    mw1p = [jnp.pad(p["Mw1"][i], ((0, 0), (0, W128 - H))) for i in range(3)]
    mb1p = [jnp.pad(row(p["Mb1"][i]), ((0, 0), (0, W128 - H))) for i in range(3)]

    zeros_n = jnp.zeros((NPAD, W128), jnp.float32)
    zeros_g = jnp.zeros((GPAD, W128), jnp.float32)

    h, t = _tc1(x, p["W0"], row(p["b0"]), mw1p[0], mb1p[0])
    for i in range(3):
        agg = _sc_gather_segsum(t, src, dst, zeros_n)
        xm, m = _tc2(t, agg, h, p["Mw2"][i], row(p["Mb2"][i]))
        xs = _sc_gather(xm, src)
        msg = _tc3(edge_attr, xs, p["A1"][i], row(p["c1"][i]), a2p[i], c2p[i], sel)
        agg2 = _sc_segsum(msg, dst, zeros_n)
        if i < 2:
            h, t = _tc4(xm, agg2, p["Wroot"][i], m, mw1p[i + 1], mb1p[i + 1])
        else:
            h = _tc4f(xm, agg2, p["Wroot"][i])
    pooled = _sc_pool(h, batch, zeros_g)
    o = _tc5(pooled, p["W1"], row(p["b1"]), p["W2"], row(p["b2"]),
             p["W3"], row(p["b3"]))
    return o.reshape(-1)


# speedup vs baseline: 3.7996x; 1.1363x over previous
import functools
import jax
import jax.numpy as jnp
from jax import lax
from jax.experimental import pallas as pl
from jax.experimental.pallas import tpu as pltpu
from jax.experimental.pallas import tpu_sc as plsc

N = 10000
E = 160000
F_IN = 128
H = 32
G = 312
W128 = 128

NB = 1000
EB = 640

NC = 2
NS = 16
NW = NC * NS
ECH = 128
ECHUNKS = E // ECH
CPW = ECHUNKS // NW
KG_G = 3
NGRP_G = CPW // KG_G
KG_S = 2
NGRP_S = CPW // KG_S
NPAD = 10112
NROW = NPAD // NS
GPAD = 384
GROW = GPAD // NS
PCH = 40
PNCH = N // PCH
PITER = (PNCH + NW - 1) // NW


def _dot(a, b):
    return jax.lax.dot_general(a.astype(jnp.bfloat16), b.astype(jnp.bfloat16),
                               (((1,), (0,)), ((), ())),
                               preferred_element_type=jnp.float32)


def _elu(x):
    return jnp.where(x > 0, x, jnp.exp(jnp.minimum(x, 0.0)) - 1.0)


def _pad_lanes(a, nb):
    return jnp.concatenate([a, jnp.zeros((nb, W128 - H), jnp.float32)], axis=1)


def _tc1_body(x_ref, w0_ref, b0_ref, mw1_ref, mb1_ref, h_ref, t_ref):
    h = _dot(x_ref[...], w0_ref[...]) + b0_ref[...]
    h_ref[...] = h
    t_ref[...] = jnp.maximum(_dot(h, mw1_ref[...]) + mb1_ref[...], 0.0)


def _tc1(x, w0, b0, mw1p, mb1p):
    grid = (N // NB,)
    full = lambda a: pl.BlockSpec(a.shape, lambda i: (0,) * a.ndim)
    return pl.pallas_call(
        _tc1_body,
        grid=grid,
        in_specs=[pl.BlockSpec((NB, F_IN), lambda i: (i, 0)),
                  full(w0), full(b0), full(mw1p), full(mb1p)],
        out_specs=[pl.BlockSpec((NB, H), lambda i: (i, 0)),
                   pl.BlockSpec((NB, W128), lambda i: (i, 0))],
        out_shape=[jax.ShapeDtypeStruct((N, H), jnp.float32),
                   jax.ShapeDtypeStruct((N, W128), jnp.float32)],
    )(x, w0, b0, mw1p, mb1p)


def _tc2_body(t_ref, agg_ref, h_ref, mw2_ref, mb2_ref, xm_ref, m_ref):
    aggs = agg_ref[...]
    agg = (aggs[0] + aggs[1])[:, :H]
    t = t_ref[...][:, :H]
    m = jax.nn.sigmoid(_dot(t + agg, mw2_ref[...]) + mb2_ref[...])
    m_ref[...] = m
    xm_ref[...] = _pad_lanes(h_ref[...] * m, NB)


def _tc2(t, agg, h, mw2, mb2):
    grid = (N // NB,)
    full = lambda a: pl.BlockSpec(a.shape, lambda i: (0,) * a.ndim)
    return pl.pallas_call(
        _tc2_body,
        grid=grid,
        in_specs=[pl.BlockSpec((NB, W128), lambda i: (i, 0)),
                  pl.BlockSpec((NC, NB, W128), lambda i: (0, i, 0)),
                  pl.BlockSpec((NB, H), lambda i: (i, 0)),
                  full(mw2), full(mb2)],
        out_specs=[pl.BlockSpec((NB, W128), lambda i: (i, 0)),
                   pl.BlockSpec((NB, 1), lambda i: (i, 0))],
        out_shape=[jax.ShapeDtypeStruct((N, W128), jnp.float32),
                   jax.ShapeDtypeStruct((N, 1), jnp.float32)],
    )(t, agg, h, mw2, mb2)


def _tc3_body(ea_ref, xs_ref, a1_ref, c1_ref, a2p_ref, c2p_ref, sel_ref, msg_ref):
    u = jnp.maximum(_dot(ea_ref[...], a1_ref[...]) + c1_ref[...], 0.0)
    w = _dot(u, a2p_ref[...]) + c2p_ref[...]
    xs = xs_ref[...][:, :H]
    xt = jnp.concatenate([xs] * H, axis=1)
    prod = (w.astype(jnp.bfloat16).astype(jnp.float32)
            * xt.astype(jnp.bfloat16).astype(jnp.float32))
    ph = prod.astype(jnp.bfloat16)
    pr = (prod - ph.astype(jnp.float32)).astype(jnp.bfloat16)
    selb = sel_ref[...].astype(jnp.bfloat16)
    dims = (((1,), (0,)), ((), ()))
    msg = (jax.lax.dot_general(ph, selb, dims, preferred_element_type=jnp.float32)
           + jax.lax.dot_general(pr, selb, dims, preferred_element_type=jnp.float32))
    msg_ref[...] = _pad_lanes(msg, EB)


def _tc3(ea, xs, a1, c1, a2p, c2p, sel):
    grid = (E // EB,)
    full = lambda a: pl.BlockSpec(a.shape, lambda i: (0,) * a.ndim)
    return pl.pallas_call(
        _tc3_body,
        grid=grid,
        in_specs=[pl.BlockSpec((EB, 5), lambda i: (i, 0)),
                  pl.BlockSpec((EB, W128), lambda i: (i, 0)),
                  full(a1), full(c1), full(a2p), full(c2p), full(sel)],
        out_specs=pl.BlockSpec((EB, W128), lambda i: (i, 0)),
        out_shape=jax.ShapeDtypeStruct((E, W128), jnp.float32),
    )(ea, xs, a1, c1, a2p, c2p, sel)


def _tc4_body(xm_ref, agg_ref, wroot_ref, m_ref, mw1_ref, mb1_ref, h_ref, t_ref):
    aggs = agg_ref[...]
    agg = (aggs[0] + aggs[1])[:, :H]
    h = _elu(_dot(xm_ref[...][:, :H], wroot_ref[...]) + agg)
    h_ref[...] = h
    hm = h * m_ref[...]
    t_ref[...] = jnp.maximum(_dot(hm, mw1_ref[...]) + mb1_ref[...], 0.0)


def _tc4(xm, agg, wroot, m, mw1p, mb1p):
    grid = (N // NB,)
    full = lambda a: pl.BlockSpec(a.shape, lambda i: (0,) * a.ndim)
    return pl.pallas_call(
        _tc4_body,
        grid=grid,
        in_specs=[pl.BlockSpec((NB, W128), lambda i: (i, 0)),
                  pl.BlockSpec((NC, NB, W128), lambda i: (0, i, 0)),
                  full(wroot),
                  pl.BlockSpec((NB, 1), lambda i: (i, 0)),
                  full(mw1p), full(mb1p)],
        out_specs=[pl.BlockSpec((NB, H), lambda i: (i, 0)),
                   pl.BlockSpec((NB, W128), lambda i: (i, 0))],
        out_shape=[jax.ShapeDtypeStruct((N, H), jnp.float32),
                   jax.ShapeDtypeStruct((N, W128), jnp.float32)],
    )(xm, agg, wroot, m, mw1p, mb1p)


def _tc4f_body(xm_ref, agg_ref, wroot_ref, h_ref):
    aggs = agg_ref[...]
    agg = (aggs[0] + aggs[1])[:, :H]
    h_ref[...] = _pad_lanes(_elu(_dot(xm_ref[...][:, :H], wroot_ref[...]) + agg), NB)


def _tc4f(xm, agg, wroot):
    grid = (N // NB,)
    full = lambda a: pl.BlockSpec(a.shape, lambda i: (0,) * a.ndim)
    return pl.pallas_call(
        _tc4f_body,
        grid=grid,
        in_specs=[pl.BlockSpec((NB, W128), lambda i: (i, 0)),
                  pl.BlockSpec((NC, NB, W128), lambda i: (0, i, 0)),
                  full(wroot)],
        out_specs=pl.BlockSpec((NB, W128), lambda i: (i, 0)),
        out_shape=jax.ShapeDtypeStruct((N, W128), jnp.float32),
    )(xm, agg, wroot)


def _tc5_body(p_ref, w1_ref, b1_ref, w2_ref, b2_ref, w3_ref, b3_ref, o_ref):
    ps = p_ref[...]
    pooled = (ps[0] + ps[1])[:G, :H]
    o = _elu(_dot(pooled, w1_ref[...]) + b1_ref[...])
    o = _elu(_dot(o, w2_ref[...]) + b2_ref[...])
    o_ref[...] = _dot(o, w3_ref[...]) + b3_ref[...]


def _tc5(pooled, w1, b1, w2, b2, w3, b3):
    full = lambda a: pl.BlockSpec(a.shape, lambda *_: (0,) * a.ndim)
    return pl.pallas_call(
        _tc5_body,
        in_specs=[full(pooled), full(w1), full(b1), full(w2), full(b2),
                  full(w3), full(b3)],
        out_specs=full(jnp.zeros((G, 1))),
        out_shape=jax.ShapeDtypeStruct((G, 1), jnp.float32),
    )(pooled, w1, b1, w2, b2, w3, b3)


def _sc_mesh():
    return plsc.VectorSubcoreMesh(core_axis_name="c", subcore_axis_name="s")


def _worker_chunks():
    cid = lax.axis_index("c")
    sid = lax.axis_index("s")
    wid = cid * NS + sid
    cbase = wid * CPW + jnp.minimum(wid, 2)
    return cid, sid, wid, cbase, wid < 2


def _load_idx(sd_hbm, idxb, cbase, extra):
    pltpu.sync_copy(sd_hbm.at[pl.ds(cbase, CPW)], idxb.at[pl.ds(0, CPW)])

    @pl.when(extra)
    def _():
        pltpu.sync_copy(sd_hbm.at[pl.ds(cbase + CPW, 1)],
                        idxb.at[pl.ds(CPW, 1)])


@functools.partial(
    pl.kernel,
    out_type=jax.ShapeDtypeStruct((NC, NPAD, W128), jnp.float32),
    mesh=_sc_mesh(),
    scratch_types=[
        pltpu.VMEM((CPW + 1, 2, ECH), jnp.int32),
        pltpu.VMEM((KG_S, ECH, W128), jnp.float32),
        pltpu.VMEM_SHARED((NPAD, W128), jnp.float32),
        pltpu.SemaphoreType.DMA,
    ],
)
def _sc_gather_segsum(t_hbm, sd_hbm, zeros_hbm, out_hbm, idxb, rows, acc, sem):
    cid, sid, wid, cbase, extra = _worker_chunks()
    rbase = sid * NROW
    _load_idx(sd_hbm, idxb, cbase, extra)
    pltpu.sync_copy(zeros_hbm.at[pl.ds(rbase, NROW)],
                    acc.at[pl.ds(rbase, NROW)])
    plsc.subcore_barrier()

    def body(g, carry):
        c0 = g * KG_S
        cps = [pltpu.async_copy(t_hbm.at[idxb.at[c0 + b, 0]], rows.at[b], sem)
               for b in range(KG_S)]
        for cp in cps:
            cp.wait()
        for b in range(KG_S):
            pltpu.sync_copy(rows.at[b], acc.at[idxb.at[c0 + b, 1]], add=True)
        return carry

    lax.fori_loop(0, NGRP_S, body, 0)
    c_last = NGRP_S * KG_S
    pltpu.async_copy(t_hbm.at[idxb.at[c_last, 0]], rows.at[0], sem).wait()
    pltpu.sync_copy(rows.at[0], acc.at[idxb.at[c_last, 1]], add=True)

    @pl.when(extra)
    def _():
        pltpu.async_copy(t_hbm.at[idxb.at[CPW, 0]], rows.at[0], sem).wait()
        pltpu.sync_copy(rows.at[0], acc.at[idxb.at[CPW, 1]], add=True)

    plsc.subcore_barrier()
    pltpu.sync_copy(acc.at[pl.ds(rbase, NROW)],
                    out_hbm.at[cid].at[pl.ds(rbase, NROW)])


@functools.partial(
    pl.kernel,
    out_type=jax.ShapeDtypeStruct((E, W128), jnp.float32),
    mesh=_sc_mesh(),
    scratch_types=[
        pltpu.VMEM((CPW + 1, 2, ECH), jnp.int32),
        pltpu.VMEM((KG_G, ECH, W128), jnp.float32),
        pltpu.SemaphoreType.DMA,
    ],
)
def _sc_gather(t_hbm, sd_hbm, out_hbm, idxb, rows, sem):
    cid, sid, wid, cbase, extra = _worker_chunks()
    _load_idx(sd_hbm, idxb, cbase, extra)

    def body(g, carry):
        c0 = g * KG_G
        cps = [pltpu.async_copy(t_hbm.at[idxb.at[c0 + b, 0]], rows.at[b], sem)
               for b in range(KG_G)]
        for cp in cps:
            cp.wait()
        for b in range(KG_G):
            pltpu.sync_copy(rows.at[b],
                            out_hbm.at[pl.ds((cbase + c0 + b) * ECH, ECH)])
        return carry

    lax.fori_loop(0, NGRP_G, body, 0)

    @pl.when(extra)
    def _():
        pltpu.async_copy(t_hbm.at[idxb.at[CPW, 0]], rows.at[0], sem).wait()
        pltpu.sync_copy(rows.at[0],
                        out_hbm.at[pl.ds((cbase + CPW) * ECH, ECH)])


@functools.partial(
    pl.kernel,
    out_type=jax.ShapeDtypeStruct((NC, NPAD, W128), jnp.float32),
    mesh=_sc_mesh(),
    scratch_types=[
        pltpu.VMEM((CPW + 1, 2, ECH), jnp.int32),
        pltpu.VMEM((KG_S, ECH, W128), jnp.float32),
        pltpu.VMEM_SHARED((NPAD, W128), jnp.float32),
        pltpu.SemaphoreType.DMA,
    ],
)
def _sc_segsum(vals_hbm, sd_hbm, zeros_hbm, out_hbm, idxb, rows, acc, sem):
    cid, sid, wid, cbase, extra = _worker_chunks()
    rbase = sid * NROW
    _load_idx(sd_hbm, idxb, cbase, extra)
    pltpu.sync_copy(zeros_hbm.at[pl.ds(rbase, NROW)],
                    acc.at[pl.ds(rbase, NROW)])
    plsc.subcore_barrier()

    def body(g, carry):
        c0 = g * KG_S
        cps = [pltpu.async_copy(
                   vals_hbm.at[pl.ds((cbase + c0 + b) * ECH, ECH)],
                   rows.at[b], sem)
               for b in range(KG_S)]
        for cp in cps:
            cp.wait()
        for b in range(KG_S):
            pltpu.sync_copy(rows.at[b], acc.at[idxb.at[c0 + b, 1]], add=True)
        return carry

    lax.fori_loop(0, NGRP_S, body, 0)
    c_last = NGRP_S * KG_S
    pltpu.async_copy(vals_hbm.at[pl.ds((cbase + c_last) * ECH, ECH)],
                     rows.at[0], sem).wait()
    pltpu.sync_copy(rows.at[0], acc.at[idxb.at[c_last, 1]], add=True)

    @pl.when(extra)
    def _():
        pltpu.async_copy(vals_hbm.at[pl.ds((cbase + CPW) * ECH, ECH)],
                         rows.at[0], sem).wait()
        pltpu.sync_copy(rows.at[0], acc.at[idxb.at[CPW, 1]], add=True)

    plsc.subcore_barrier()
    pltpu.sync_copy(acc.at[pl.ds(rbase, NROW)],
                    out_hbm.at[cid].at[pl.ds(rbase, NROW)])


@functools.partial(
    pl.kernel,
    out_type=jax.ShapeDtypeStruct((NC, GPAD, W128), jnp.float32),
    mesh=_sc_mesh(),
    scratch_types=[
        pltpu.VMEM((PCH,), jnp.int32),
        pltpu.VMEM((PCH, W128), jnp.float32),
        pltpu.VMEM_SHARED((GPAD, W128), jnp.float32),
    ],
)
def _sc_pool(h_hbm, batch_hbm, zeros_hbm, out_hbm, bidx, rows, acc):
    cid = lax.axis_index("c")
    sid = lax.axis_index("s")
    wid = cid * NS + sid
    rbase = sid * GROW

    pltpu.sync_copy(zeros_hbm.at[pl.ds(rbase, GROW)],
                    acc.at[pl.ds(rbase, GROW)])
    plsc.subcore_barrier()

    def body(k, carry):
        c = wid + k * NW

        @pl.when(c < PNCH)
        def _():
            base = c * PCH
            pltpu.sync_copy(batch_hbm.at[pl.ds(base, PCH)], bidx)
            pltpu.sync_copy(h_hbm.at[pl.ds(base, PCH)], rows)
            pltpu.sync_copy(rows, acc.at[bidx], add=True)

        return carry

    lax.fori_loop(0, PITER, body, 0)
    plsc.subcore_barrier()
    pltpu.sync_copy(acc.at[pl.ds(rbase, GROW)],
                    out_hbm.at[cid].at[pl.ds(rbase, GROW)])


def kernel(x, edge_index, edge_attr, batch, params):
    sd = jnp.stack([edge_index[0].reshape(ECHUNKS, ECH),
                    edge_index[1].reshape(ECHUNKS, ECH)], axis=1)
    p = params
    row = lambda v: v.reshape(1, -1)

    a2p = [p["A2"][i].reshape(F_IN, H, H).transpose(0, 2, 1).reshape(F_IN, H * H)
           for i in range(3)]
    c2p = [p["c2"][i].reshape(H, H).T.reshape(1, H * H) for i in range(3)]
    sel = (jnp.arange(H * H, dtype=jnp.int32)[:, None] // H
           == jnp.arange(H, dtype=jnp.int32)[None, :]).astype(jnp.float32)
    mw1p = [jnp.pad(p["Mw1"][i], ((0, 0), (0, W128 - H))) for i in range(3)]
    mb1p = [jnp.pad(row(p["Mb1"][i]), ((0, 0), (0, W128 - H))) for i in range(3)]

    zeros_n = jnp.zeros((NPAD, W128), jnp.float32)
    zeros_g = jnp.zeros((GPAD, W128), jnp.float32)

    h, t = _tc1(x, p["W0"], row(p["b0"]), mw1p[0], mb1p[0])
    for i in range(3):
        agg = _sc_gather_segsum(t, sd, zeros_n)
        xm, m = _tc2(t, agg, h, p["Mw2"][i], row(p["Mb2"][i]))
        xs = _sc_gather(xm, sd)
        msg = _tc3(edge_attr, xs, p["A1"][i], row(p["c1"][i]), a2p[i], c2p[i], sel)
        agg2 = _sc_segsum(msg, sd, zeros_n)
        if i < 2:
            h, t = _tc4(xm, agg2, p["Wroot"][i], m, mw1p[i + 1], mb1p[i + 1])
        else:
            h = _tc4f(xm, agg2, p["Wroot"][i])
    pooled = _sc_pool(h, batch, zeros_g)
    o = _tc5(pooled, p["W1"], row(p["b1"]), p["W2"], row(p["b2"]),
             p["W3"], row(p["b3"]))
    return o.reshape(-1)


# speedup vs baseline: 4.0630x; 1.0693x over previous
import functools
import jax
import jax.numpy as jnp
from jax import lax
from jax.experimental import pallas as pl
from jax.experimental.pallas import tpu as pltpu
from jax.experimental.pallas import tpu_sc as plsc

N = 10000
E = 160000
F_IN = 128
H = 32
G = 312
W128 = 128

NB = 1000
EB = 640

NC = 2
NS = 16
NW = NC * NS
ECH = 128
ECHUNKS = E // ECH
CPW = ECHUNKS // NW
KG_G = 3
NGRP_G = CPW // KG_G
KG_S = 2
NGRP_S = CPW // KG_S
NPAD = 10112
NROW = NPAD // NS
GPAD = 384
GROW = GPAD // NS
PCH = 40
PNCH = N // PCH
PITER = (PNCH + NW - 1) // NW


def _dot(a, b):
    return jax.lax.dot_general(a.astype(jnp.bfloat16), b.astype(jnp.bfloat16),
                               (((1,), (0,)), ((), ())),
                               preferred_element_type=jnp.float32)


def _elu(x):
    return jnp.where(x > 0, x, jnp.exp(jnp.minimum(x, 0.0)) - 1.0)


def _pad_lanes(a, nb):
    return jnp.concatenate([a, jnp.zeros((nb, W128 - H), jnp.float32)], axis=1)


def _tc1_body(x_ref, w0_ref, b0_ref, mw1_ref, mb1_ref, h_ref, t_ref):
    h = _dot(x_ref[...], w0_ref[...]) + b0_ref[...]
    h_ref[...] = h
    t_ref[...] = jnp.maximum(_dot(h, mw1_ref[...]) + mb1_ref[...], 0.0)


def _tc1(x, w0, b0, mw1p, mb1p):
    grid = (N // NB,)
    full = lambda a: pl.BlockSpec(a.shape, lambda i: (0,) * a.ndim)
    return pl.pallas_call(
        _tc1_body,
        grid=grid,
        in_specs=[pl.BlockSpec((NB, F_IN), lambda i: (i, 0)),
                  full(w0), full(b0), full(mw1p), full(mb1p)],
        out_specs=[pl.BlockSpec((NB, H), lambda i: (i, 0)),
                   pl.BlockSpec((NB, W128), lambda i: (i, 0))],
        out_shape=[jax.ShapeDtypeStruct((N, H), jnp.float32),
                   jax.ShapeDtypeStruct((N, W128), jnp.float32)],
    )(x, w0, b0, mw1p, mb1p)


def _tc2_body(t_ref, agg_ref, h_ref, mw2_ref, mb2_ref, xm_ref, m_ref):
    aggs = agg_ref[...]
    agg = (aggs[0] + aggs[1])[:, :H]
    t = t_ref[...][:, :H]
    m = jax.nn.sigmoid(_dot(t + agg, mw2_ref[...]) + mb2_ref[...])
    m_ref[...] = m
    xm_ref[...] = _pad_lanes(h_ref[...] * m, NB)


def _tc2(t, agg, h, mw2, mb2):
    grid = (N // NB,)
    full = lambda a: pl.BlockSpec(a.shape, lambda i: (0,) * a.ndim)
    return pl.pallas_call(
        _tc2_body,
        grid=grid,
        in_specs=[pl.BlockSpec((NB, W128), lambda i: (i, 0)),
                  pl.BlockSpec((NC, NB, W128), lambda i: (0, i, 0)),
                  pl.BlockSpec((NB, H), lambda i: (i, 0)),
                  full(mw2), full(mb2)],
        out_specs=[pl.BlockSpec((NB, W128), lambda i: (i, 0)),
                   pl.BlockSpec((NB, 1), lambda i: (i, 0))],
        out_shape=[jax.ShapeDtypeStruct((N, W128), jnp.float32),
                   jax.ShapeDtypeStruct((N, 1), jnp.float32)],
    )(t, agg, h, mw2, mb2)


def _tc3_body(ea_ref, xs_ref, a1_ref, c1_ref, a2p_ref, c2p_ref, sel_ref, msg_ref):
    u = jnp.maximum(_dot(ea_ref[...], a1_ref[...]) + c1_ref[...], 0.0)
    w = _dot(u, a2p_ref[...]) + c2p_ref[...]
    xs = xs_ref[...][:, :H]
    xt = jnp.concatenate([xs] * H, axis=1)
    prod = (w.astype(jnp.bfloat16).astype(jnp.float32)
            * xt.astype(jnp.bfloat16).astype(jnp.float32))
    ph = prod.astype(jnp.bfloat16)
    pr = (prod - ph.astype(jnp.float32)).astype(jnp.bfloat16)
    selb = sel_ref[...].astype(jnp.bfloat16)
    dims = (((1,), (0,)), ((), ()))
    msg = (jax.lax.dot_general(ph, selb, dims, preferred_element_type=jnp.float32)
           + jax.lax.dot_general(pr, selb, dims, preferred_element_type=jnp.float32))
    msg_ref[...] = _pad_lanes(msg, EB)


def _tc3(ea, xs, a1, c1, a2p, c2p, sel):
    grid = (E // EB,)
    full = lambda a: pl.BlockSpec(a.shape, lambda i: (0,) * a.ndim)
    return pl.pallas_call(
        _tc3_body,
        grid=grid,
        in_specs=[pl.BlockSpec((EB, 5), lambda i: (i, 0)),
                  pl.BlockSpec((EB, W128), lambda i: (i, 0)),
                  full(a1), full(c1), full(a2p), full(c2p), full(sel)],
        out_specs=pl.BlockSpec((EB, W128), lambda i: (i, 0)),
        out_shape=jax.ShapeDtypeStruct((E, W128), jnp.float32),
    )(ea, xs, a1, c1, a2p, c2p, sel)


def _tc4_body(xm_ref, agg_ref, wroot_ref, m_ref, mw1_ref, mb1_ref, h_ref, t_ref):
    aggs = agg_ref[...]
    agg = (aggs[0] + aggs[1])[:, :H]
    h = _elu(_dot(xm_ref[...][:, :H], wroot_ref[...]) + agg)
    h_ref[...] = h
    hm = h * m_ref[...]
    t_ref[...] = jnp.maximum(_dot(hm, mw1_ref[...]) + mb1_ref[...], 0.0)


def _tc4(xm, agg, wroot, m, mw1p, mb1p):
    grid = (N // NB,)
    full = lambda a: pl.BlockSpec(a.shape, lambda i: (0,) * a.ndim)
    return pl.pallas_call(
        _tc4_body,
        grid=grid,
        in_specs=[pl.BlockSpec((NB, W128), lambda i: (i, 0)),
                  pl.BlockSpec((NC, NB, W128), lambda i: (0, i, 0)),
                  full(wroot),
                  pl.BlockSpec((NB, 1), lambda i: (i, 0)),
                  full(mw1p), full(mb1p)],
        out_specs=[pl.BlockSpec((NB, H), lambda i: (i, 0)),
                   pl.BlockSpec((NB, W128), lambda i: (i, 0))],
        out_shape=[jax.ShapeDtypeStruct((N, H), jnp.float32),
                   jax.ShapeDtypeStruct((N, W128), jnp.float32)],
    )(xm, agg, wroot, m, mw1p, mb1p)


def _tc4f_body(xm_ref, agg_ref, wroot_ref, h_ref):
    aggs = agg_ref[...]
    agg = (aggs[0] + aggs[1])[:, :H]
    h_ref[...] = _pad_lanes(_elu(_dot(xm_ref[...][:, :H], wroot_ref[...]) + agg), NB)


def _tc4f(xm, agg, wroot):
    grid = (N // NB,)
    full = lambda a: pl.BlockSpec(a.shape, lambda i: (0,) * a.ndim)
    return pl.pallas_call(
        _tc4f_body,
        grid=grid,
        in_specs=[pl.BlockSpec((NB, W128), lambda i: (i, 0)),
                  pl.BlockSpec((NC, NB, W128), lambda i: (0, i, 0)),
                  full(wroot)],
        out_specs=pl.BlockSpec((NB, W128), lambda i: (i, 0)),
        out_shape=jax.ShapeDtypeStruct((N, W128), jnp.float32),
    )(xm, agg, wroot)


def _tc5_body(p_ref, w1_ref, b1_ref, w2_ref, b2_ref, w3_ref, b3_ref, o_ref):
    ps = p_ref[...]
    pooled = (ps[0] + ps[1])[:G, :H]
    o = _elu(_dot(pooled, w1_ref[...]) + b1_ref[...])
    o = _elu(_dot(o, w2_ref[...]) + b2_ref[...])
    o_ref[...] = _dot(o, w3_ref[...]) + b3_ref[...]


def _tc5(pooled, w1, b1, w2, b2, w3, b3):
    full = lambda a: pl.BlockSpec(a.shape, lambda *_: (0,) * a.ndim)
    return pl.pallas_call(
        _tc5_body,
        in_specs=[full(pooled), full(w1), full(b1), full(w2), full(b2),
                  full(w3), full(b3)],
        out_specs=full(jnp.zeros((G, 1))),
        out_shape=jax.ShapeDtypeStruct((G, 1), jnp.float32),
    )(pooled, w1, b1, w2, b2, w3, b3)


def _sc_mesh():
    return plsc.VectorSubcoreMesh(core_axis_name="c", subcore_axis_name="s")


def _worker_chunks():
    cid = lax.axis_index("c")
    sid = lax.axis_index("s")
    wid = cid * NS + sid
    cbase = wid * CPW + jnp.minimum(wid, 2)
    return cid, sid, wid, cbase, wid < 2


def _load_idx(sd_hbm, idxb, cbase, extra):
    pltpu.sync_copy(sd_hbm.at[pl.ds(cbase, CPW)], idxb.at[pl.ds(0, CPW)])

    @pl.when(extra)
    def _():
        pltpu.sync_copy(sd_hbm.at[pl.ds(cbase + CPW, 1)],
                        idxb.at[pl.ds(CPW, 1)])


@functools.partial(
    pl.kernel,
    out_type=jax.ShapeDtypeStruct((NC, NPAD, W128), jnp.float32),
    mesh=_sc_mesh(),
    scratch_types=[
        pltpu.VMEM((CPW + 1, 2, ECH), jnp.int32),
        pltpu.VMEM((KG_S, ECH, W128), jnp.float32),
        pltpu.VMEM_SHARED((NPAD, W128), jnp.float32),
        pltpu.SemaphoreType.DMA,
        pltpu.SemaphoreType.DMA,
    ],
)
def _sc_gather_segsum(t_hbm, sd_hbm, zeros_hbm, out_hbm, idxb, rows, acc,
                      sem0, sem1):
    cid, sid, wid, cbase, extra = _worker_chunks()
    rbase = sid * NROW
    sems = (sem0, sem1)
    _load_idx(sd_hbm, idxb, cbase, extra)
    pltpu.sync_copy(zeros_hbm.at[pl.ds(rbase, NROW)],
                    acc.at[pl.ds(rbase, NROW)])
    plsc.subcore_barrier()

    for b in range(KG_S):
        pltpu.async_copy(t_hbm.at[idxb.at[b, 0]], rows.at[b], sems[b])

    def body(g, carry):
        c0 = g * KG_S
        for b in range(KG_S):
            pltpu.make_async_copy(t_hbm.at[idxb.at[c0 + b, 0]],
                                  rows.at[b], sems[b]).wait()
            pltpu.sync_copy(rows.at[b], acc.at[idxb.at[c0 + b, 1]], add=True)

            @pl.when(g < NGRP_S - 1)
            def _():
                pltpu.async_copy(t_hbm.at[idxb.at[c0 + KG_S + b, 0]],
                                 rows.at[b], sems[b])

        return carry

    lax.fori_loop(0, NGRP_S, body, 0)
    c_last = NGRP_S * KG_S
    pltpu.async_copy(t_hbm.at[idxb.at[c_last, 0]], rows.at[0], sem0).wait()
    pltpu.sync_copy(rows.at[0], acc.at[idxb.at[c_last, 1]], add=True)

    @pl.when(extra)
    def _():
        pltpu.async_copy(t_hbm.at[idxb.at[CPW, 0]], rows.at[0], sem0).wait()
        pltpu.sync_copy(rows.at[0], acc.at[idxb.at[CPW, 1]], add=True)

    plsc.subcore_barrier()
    pltpu.sync_copy(acc.at[pl.ds(rbase, NROW)],
                    out_hbm.at[cid].at[pl.ds(rbase, NROW)])


@functools.partial(
    pl.kernel,
    out_type=jax.ShapeDtypeStruct((E, W128), jnp.float32),
    mesh=_sc_mesh(),
    scratch_types=[
        pltpu.VMEM((CPW + 1, 2, ECH), jnp.int32),
        pltpu.VMEM((KG_G, ECH, W128), jnp.float32),
        pltpu.SemaphoreType.DMA,
        pltpu.SemaphoreType.DMA,
        pltpu.SemaphoreType.DMA,
    ],
)
def _sc_gather(t_hbm, sd_hbm, out_hbm, idxb, rows, sem0, sem1, sem2):
    cid, sid, wid, cbase, extra = _worker_chunks()
    sems = (sem0, sem1, sem2)
    _load_idx(sd_hbm, idxb, cbase, extra)

    for b in range(KG_G):
        pltpu.async_copy(t_hbm.at[idxb.at[b, 0]], rows.at[b], sems[b])

    def body(g, carry):
        c0 = g * KG_G
        for b in range(KG_G):
            pltpu.make_async_copy(t_hbm.at[idxb.at[c0 + b, 0]],
                                  rows.at[b], sems[b]).wait()
            pltpu.sync_copy(rows.at[b],
                            out_hbm.at[pl.ds((cbase + c0 + b) * ECH, ECH)])

            @pl.when(g < NGRP_G - 1)
            def _():
                pltpu.async_copy(t_hbm.at[idxb.at[c0 + KG_G + b, 0]],
                                 rows.at[b], sems[b])

        return carry

    lax.fori_loop(0, NGRP_G, body, 0)

    @pl.when(extra)
    def _():
        pltpu.async_copy(t_hbm.at[idxb.at[CPW, 0]], rows.at[0], sem0).wait()
        pltpu.sync_copy(rows.at[0],
                        out_hbm.at[pl.ds((cbase + CPW) * ECH, ECH)])


@functools.partial(
    pl.kernel,
    out_type=jax.ShapeDtypeStruct((NC, NPAD, W128), jnp.float32),
    mesh=_sc_mesh(),
    scratch_types=[
        pltpu.VMEM((CPW + 1, 2, ECH), jnp.int32),
        pltpu.VMEM((KG_S, ECH, W128), jnp.float32),
        pltpu.VMEM_SHARED((NPAD, W128), jnp.float32),
        pltpu.SemaphoreType.DMA,
        pltpu.SemaphoreType.DMA,
    ],
)
def _sc_segsum(vals_hbm, sd_hbm, zeros_hbm, out_hbm, idxb, rows, acc,
               sem0, sem1):
    cid, sid, wid, cbase, extra = _worker_chunks()
    rbase = sid * NROW
    sems = (sem0, sem1)
    _load_idx(sd_hbm, idxb, cbase, extra)
    pltpu.sync_copy(zeros_hbm.at[pl.ds(rbase, NROW)],
                    acc.at[pl.ds(rbase, NROW)])
    plsc.subcore_barrier()

    for b in range(KG_S):
        pltpu.async_copy(vals_hbm.at[pl.ds((cbase + b) * ECH, ECH)],
                         rows.at[b], sems[b])

    def body(g, carry):
        c0 = g * KG_S
        for b in range(KG_S):
            pltpu.make_async_copy(
                vals_hbm.at[pl.ds((cbase + c0 + b) * ECH, ECH)],
                rows.at[b], sems[b]).wait()
            pltpu.sync_copy(rows.at[b], acc.at[idxb.at[c0 + b, 1]], add=True)

            @pl.when(g < NGRP_S - 1)
            def _():
                pltpu.async_copy(
                    vals_hbm.at[pl.ds((cbase + c0 + KG_S + b) * ECH, ECH)],
                    rows.at[b], sems[b])

        return carry

    lax.fori_loop(0, NGRP_S, body, 0)
    c_last = NGRP_S * KG_S
    pltpu.async_copy(vals_hbm.at[pl.ds((cbase + c_last) * ECH, ECH)],
                     rows.at[0], sem0).wait()
    pltpu.sync_copy(rows.at[0], acc.at[idxb.at[c_last, 1]], add=True)

    @pl.when(extra)
    def _():
        pltpu.async_copy(vals_hbm.at[pl.ds((cbase + CPW) * ECH, ECH)],
                         rows.at[0], sem0).wait()
        pltpu.sync_copy(rows.at[0], acc.at[idxb.at[CPW, 1]], add=True)

    plsc.subcore_barrier()
    pltpu.sync_copy(acc.at[pl.ds(rbase, NROW)],
                    out_hbm.at[cid].at[pl.ds(rbase, NROW)])


@functools.partial(
    pl.kernel,
    out_type=jax.ShapeDtypeStruct((NC, GPAD, W128), jnp.float32),
    mesh=_sc_mesh(),
    scratch_types=[
        pltpu.VMEM((PCH,), jnp.int32),
        pltpu.VMEM((PCH, W128), jnp.float32),
        pltpu.VMEM_SHARED((GPAD, W128), jnp.float32),
    ],
)
def _sc_pool(h_hbm, batch_hbm, zeros_hbm, out_hbm, bidx, rows, acc):
    cid = lax.axis_index("c")
    sid = lax.axis_index("s")
    wid = cid * NS + sid
    rbase = sid * GROW

    pltpu.sync_copy(zeros_hbm.at[pl.ds(rbase, GROW)],
                    acc.at[pl.ds(rbase, GROW)])
    plsc.subcore_barrier()

    def body(k, carry):
        c = wid + k * NW

        @pl.when(c < PNCH)
        def _():
            base = c * PCH
            pltpu.sync_copy(batch_hbm.at[pl.ds(base, PCH)], bidx)
            pltpu.sync_copy(h_hbm.at[pl.ds(base, PCH)], rows)
            pltpu.sync_copy(rows, acc.at[bidx], add=True)

        return carry

    lax.fori_loop(0, PITER, body, 0)
    plsc.subcore_barrier()
    pltpu.sync_copy(acc.at[pl.ds(rbase, GROW)],
                    out_hbm.at[cid].at[pl.ds(rbase, GROW)])


def kernel(x, edge_index, edge_attr, batch, params):
    sd = jnp.stack([edge_index[0].reshape(ECHUNKS, ECH),
                    edge_index[1].reshape(ECHUNKS, ECH)], axis=1)
    p = params
    row = lambda v: v.reshape(1, -1)

    a2p = [p["A2"][i].reshape(F_IN, H, H).transpose(0, 2, 1).reshape(F_IN, H * H)
           for i in range(3)]
    c2p = [p["c2"][i].reshape(H, H).T.reshape(1, H * H) for i in range(3)]
    sel = (jnp.arange(H * H, dtype=jnp.int32)[:, None] // H
           == jnp.arange(H, dtype=jnp.int32)[None, :]).astype(jnp.float32)
    mw1p = [jnp.pad(p["Mw1"][i], ((0, 0), (0, W128 - H))) for i in range(3)]
    mb1p = [jnp.pad(row(p["Mb1"][i]), ((0, 0), (0, W128 - H))) for i in range(3)]

    zeros_n = jnp.zeros((NPAD, W128), jnp.float32)
    zeros_g = jnp.zeros((GPAD, W128), jnp.float32)

    h, t = _tc1(x, p["W0"], row(p["b0"]), mw1p[0], mb1p[0])
    for i in range(3):
        agg = _sc_gather_segsum(t, sd, zeros_n)
        xm, m = _tc2(t, agg, h, p["Mw2"][i], row(p["Mb2"][i]))
        xs = _sc_gather(xm, sd)
        msg = _tc3(edge_attr, xs, p["A1"][i], row(p["c1"][i]), a2p[i], c2p[i], sel)
        agg2 = _sc_segsum(msg, sd, zeros_n)
        if i < 2:
            h, t = _tc4(xm, agg2, p["Wroot"][i], m, mw1p[i + 1], mb1p[i + 1])
        else:
            h = _tc4f(xm, agg2, p["Wroot"][i])
    pooled = _sc_pool(h, batch, zeros_g)
    o = _tc5(pooled, p["W1"], row(p["b1"]), p["W2"], row(p["b2"]),
             p["W3"], row(p["b3"]))
    return o.reshape(-1)


# speedup vs baseline: 4.0902x; 1.0067x over previous
import functools
import jax
import jax.numpy as jnp
from jax import lax
from jax.experimental import pallas as pl
from jax.experimental.pallas import tpu as pltpu
from jax.experimental.pallas import tpu_sc as plsc

N = 10000
E = 160000
F_IN = 128
H = 32
G = 312
W128 = 128

NB = 1000
EB = 640

NC = 2
NS = 16
NW = NC * NS
ECH = 128
ECHUNKS = E // ECH
CPW = ECHUNKS // NW
KG_G = 3
NGRP_G = CPW // KG_G
KG_S = 2
NGRP_S = CPW // KG_S
NPAD = 10112
NROW = NPAD // NS
GPAD = 384
GROW = GPAD // NS
PCH = 40
PNCH = N // PCH
PITER = (PNCH + NW - 1) // NW


def _dot(a, b):
    return jax.lax.dot_general(a.astype(jnp.bfloat16), b.astype(jnp.bfloat16),
                               (((1,), (0,)), ((), ())),
                               preferred_element_type=jnp.float32)


def _elu(x):
    return jnp.where(x > 0, x, jnp.exp(jnp.minimum(x, 0.0)) - 1.0)


def _pad_lanes(a, nb):
    return jnp.concatenate([a, jnp.zeros((nb, W128 - H), jnp.float32)], axis=1)


def _tc1_body(x_ref, w0_ref, b0_ref, mw1_ref, mb1_ref, h_ref, t_ref):
    h = _dot(x_ref[...], w0_ref[...]) + b0_ref[...]
    h_ref[...] = h
    t_ref[...] = jnp.maximum(_dot(h, mw1_ref[...]) + mb1_ref[...], 0.0)


def _tc1(x, w0, b0, mw1p, mb1p):
    grid = (N // NB,)
    full = lambda a: pl.BlockSpec(a.shape, lambda i: (0,) * a.ndim)
    return pl.pallas_call(
        _tc1_body,
        grid=grid,
        in_specs=[pl.BlockSpec((NB, F_IN), lambda i: (i, 0)),
                  full(w0), full(b0), full(mw1p), full(mb1p)],
        out_specs=[pl.BlockSpec((NB, H), lambda i: (i, 0)),
                   pl.BlockSpec((NB, W128), lambda i: (i, 0))],
        out_shape=[jax.ShapeDtypeStruct((N, H), jnp.float32),
                   jax.ShapeDtypeStruct((N, W128), jnp.float32)],
    )(x, w0, b0, mw1p, mb1p)


def _tc2_body(t_ref, agg_ref, h_ref, mw2_ref, mb2_ref, xm_ref, m_ref):
    aggs = agg_ref[...]
    agg = (aggs[0] + aggs[1])[:, :H]
    t = t_ref[...][:, :H]
    m = jax.nn.sigmoid(_dot(t + agg, mw2_ref[...]) + mb2_ref[...])
    m_ref[...] = m
    xm_ref[...] = _pad_lanes(h_ref[...] * m, NB)


def _tc2(t, agg, h, mw2, mb2):
    grid = (N // NB,)
    full = lambda a: pl.BlockSpec(a.shape, lambda i: (0,) * a.ndim)
    return pl.pallas_call(
        _tc2_body,
        grid=grid,
        in_specs=[pl.BlockSpec((NB, W128), lambda i: (i, 0)),
                  pl.BlockSpec((NC, NB, W128), lambda i: (0, i, 0)),
                  pl.BlockSpec((NB, H), lambda i: (i, 0)),
                  full(mw2), full(mb2)],
        out_specs=[pl.BlockSpec((NB, W128), lambda i: (i, 0)),
                   pl.BlockSpec((NB, 1), lambda i: (i, 0))],
        out_shape=[jax.ShapeDtypeStruct((N, W128), jnp.float32),
                   jax.ShapeDtypeStruct((N, 1), jnp.float32)],
    )(t, agg, h, mw2, mb2)


def _tc3_body(ea_ref, xs_ref, a1_ref, c1_ref, a2p_ref, c2p_ref, sel_ref, msg_ref):
    u = jnp.maximum(_dot(ea_ref[...], a1_ref[...]) + c1_ref[...], 0.0)
    w = _dot(u, a2p_ref[...]) + c2p_ref[...]
    xs = xs_ref[...][:, :H]
    xt = jnp.concatenate([xs] * H, axis=1)
    prod = (w.astype(jnp.bfloat16).astype(jnp.float32)
            * xt.astype(jnp.bfloat16).astype(jnp.float32))
    ph = prod.astype(jnp.bfloat16)
    pr = (prod - ph.astype(jnp.float32)).astype(jnp.bfloat16)
    selb = sel_ref[...].astype(jnp.bfloat16)
    dims = (((1,), (0,)), ((), ()))
    msg = (jax.lax.dot_general(ph, selb, dims, preferred_element_type=jnp.float32)
           + jax.lax.dot_general(pr, selb, dims, preferred_element_type=jnp.float32))
    msg_ref[...] = _pad_lanes(msg, EB)


def _tc3(ea, xs, a1, c1, a2p, c2p, sel):
    grid = (E // EB,)
    full = lambda a: pl.BlockSpec(a.shape, lambda i: (0,) * a.ndim)
    return pl.pallas_call(
        _tc3_body,
        grid=grid,
        in_specs=[pl.BlockSpec((EB, 5), lambda i: (i, 0)),
                  pl.BlockSpec((EB, W128), lambda i: (i, 0)),
                  full(a1), full(c1), full(a2p), full(c2p), full(sel)],
        out_specs=pl.BlockSpec((EB, W128), lambda i: (i, 0)),
        out_shape=jax.ShapeDtypeStruct((E, W128), jnp.float32),
    )(ea, xs, a1, c1, a2p, c2p, sel)


def _tc4_body(xm_ref, agg_ref, wroot_ref, m_ref, mw1_ref, mb1_ref, h_ref, t_ref):
    aggs = agg_ref[...]
    agg = (aggs[0] + aggs[1])[:, :H]
    h = _elu(_dot(xm_ref[...][:, :H], wroot_ref[...]) + agg)
    h_ref[...] = h
    hm = h * m_ref[...]
    t_ref[...] = jnp.maximum(_dot(hm, mw1_ref[...]) + mb1_ref[...], 0.0)


def _tc4(xm, agg, wroot, m, mw1p, mb1p):
    grid = (N // NB,)
    full = lambda a: pl.BlockSpec(a.shape, lambda i: (0,) * a.ndim)
    return pl.pallas_call(
        _tc4_body,
        grid=grid,
        in_specs=[pl.BlockSpec((NB, W128), lambda i: (i, 0)),
                  pl.BlockSpec((NC, NB, W128), lambda i: (0, i, 0)),
                  full(wroot),
                  pl.BlockSpec((NB, 1), lambda i: (i, 0)),
                  full(mw1p), full(mb1p)],
        out_specs=[pl.BlockSpec((NB, H), lambda i: (i, 0)),
                   pl.BlockSpec((NB, W128), lambda i: (i, 0))],
        out_shape=[jax.ShapeDtypeStruct((N, H), jnp.float32),
                   jax.ShapeDtypeStruct((N, W128), jnp.float32)],
    )(xm, agg, wroot, m, mw1p, mb1p)


def _tc4f_pool_body(xm_ref, agg_ref, wroot_ref, b_ref, p_ref):
    aggs = agg_ref[...]
    agg = (aggs[0] + aggs[1])[:, :H]
    h = _elu(_dot(xm_ref[...][:, :H], wroot_ref[...]) + agg)
    onehot = (b_ref[...] == jax.lax.broadcasted_iota(jnp.int32, (NB, GPAD), 1)
              ).astype(jnp.float32)
    part = jax.lax.dot_general(onehot, h, (((0,), (0,)), ((), ())),
                               preferred_element_type=jnp.float32,
                               precision=jax.lax.Precision.HIGHEST)

    @pl.when(pl.program_id(0) == 0)
    def _():
        p_ref[...] = jnp.zeros_like(p_ref)

    p_ref[...] += part


def _tc4f_pool(xm, agg, wroot, batch2d):
    grid = (N // NB,)
    full = lambda a: pl.BlockSpec(a.shape, lambda i: (0,) * a.ndim)
    return pl.pallas_call(
        _tc4f_pool_body,
        grid=grid,
        in_specs=[pl.BlockSpec((NB, W128), lambda i: (i, 0)),
                  pl.BlockSpec((NC, NB, W128), lambda i: (0, i, 0)),
                  full(wroot),
                  pl.BlockSpec((NB, 1), lambda i: (i, 0))],
        out_specs=pl.BlockSpec((GPAD, H), lambda i: (0, 0)),
        out_shape=jax.ShapeDtypeStruct((GPAD, H), jnp.float32),
    )(xm, agg, wroot, batch2d)


def _tc5_body(p_ref, w1_ref, b1_ref, w2_ref, b2_ref, w3_ref, b3_ref, o_ref):
    pooled = p_ref[...][:G]
    o = _elu(_dot(pooled, w1_ref[...]) + b1_ref[...])
    o = _elu(_dot(o, w2_ref[...]) + b2_ref[...])
    o_ref[...] = _dot(o, w3_ref[...]) + b3_ref[...]


def _tc5(pooled, w1, b1, w2, b2, w3, b3):
    full = lambda a: pl.BlockSpec(a.shape, lambda *_: (0,) * a.ndim)
    return pl.pallas_call(
        _tc5_body,
        in_specs=[full(pooled), full(w1), full(b1), full(w2), full(b2),
                  full(w3), full(b3)],
        out_specs=full(jnp.zeros((G, 1))),
        out_shape=jax.ShapeDtypeStruct((G, 1), jnp.float32),
    )(pooled, w1, b1, w2, b2, w3, b3)


def _sc_mesh():
    return plsc.VectorSubcoreMesh(core_axis_name="c", subcore_axis_name="s")


def _worker_chunks():
    cid = lax.axis_index("c")
    sid = lax.axis_index("s")
    wid = cid * NS + sid
    cbase = wid * CPW + jnp.minimum(wid, 2)
    return cid, sid, wid, cbase, wid < 2


def _load_idx(sd_hbm, idxb, cbase, extra):
    pltpu.sync_copy(sd_hbm.at[pl.ds(cbase, CPW)], idxb.at[pl.ds(0, CPW)])

    @pl.when(extra)
    def _():
        pltpu.sync_copy(sd_hbm.at[pl.ds(cbase + CPW, 1)],
                        idxb.at[pl.ds(CPW, 1)])


@functools.partial(
    pl.kernel,
    out_type=jax.ShapeDtypeStruct((NC, NPAD, W128), jnp.float32),
    mesh=_sc_mesh(),
    scratch_types=[
        pltpu.VMEM((CPW + 1, 2, ECH), jnp.int32),
        pltpu.VMEM((KG_S, ECH, W128), jnp.float32),
        pltpu.VMEM_SHARED((NPAD, W128), jnp.float32),
        pltpu.SemaphoreType.DMA,
        pltpu.SemaphoreType.DMA,
    ],
)
def _sc_gather_segsum(t_hbm, sd_hbm, zeros_hbm, out_hbm, idxb, rows, acc,
                      sem0, sem1):
    cid, sid, wid, cbase, extra = _worker_chunks()
    rbase = sid * NROW
    sems = (sem0, sem1)
    _load_idx(sd_hbm, idxb, cbase, extra)
    pltpu.sync_copy(zeros_hbm.at[pl.ds(rbase, NROW)],
                    acc.at[pl.ds(rbase, NROW)])
    plsc.subcore_barrier()

    for b in range(KG_S):
        pltpu.async_copy(t_hbm.at[idxb.at[b, 0]], rows.at[b], sems[b])

    def body(g, carry):
        c0 = g * KG_S
        for b in range(KG_S):
            pltpu.make_async_copy(t_hbm.at[idxb.at[c0 + b, 0]],
                                  rows.at[b], sems[b]).wait()
            pltpu.sync_copy(rows.at[b], acc.at[idxb.at[c0 + b, 1]], add=True)

            @pl.when(g < NGRP_S - 1)
            def _():
                pltpu.async_copy(t_hbm.at[idxb.at[c0 + KG_S + b, 0]],
                                 rows.at[b], sems[b])

        return carry

    lax.fori_loop(0, NGRP_S, body, 0)
    c_last = NGRP_S * KG_S
    pltpu.async_copy(t_hbm.at[idxb.at[c_last, 0]], rows.at[0], sem0).wait()
    pltpu.sync_copy(rows.at[0], acc.at[idxb.at[c_last, 1]], add=True)

    @pl.when(extra)
    def _():
        pltpu.async_copy(t_hbm.at[idxb.at[CPW, 0]], rows.at[0], sem0).wait()
        pltpu.sync_copy(rows.at[0], acc.at[idxb.at[CPW, 1]], add=True)

    plsc.subcore_barrier()
    pltpu.sync_copy(acc.at[pl.ds(rbase, NROW)],
                    out_hbm.at[cid].at[pl.ds(rbase, NROW)])


@functools.partial(
    pl.kernel,
    out_type=jax.ShapeDtypeStruct((E, W128), jnp.float32),
    mesh=_sc_mesh(),
    scratch_types=[
        pltpu.VMEM((CPW + 1, 2, ECH), jnp.int32),
        pltpu.VMEM((KG_G, ECH, W128), jnp.float32),
        pltpu.SemaphoreType.DMA,
        pltpu.SemaphoreType.DMA,
        pltpu.SemaphoreType.DMA,
    ],
)
def _sc_gather(t_hbm, sd_hbm, out_hbm, idxb, rows, sem0, sem1, sem2):
    cid, sid, wid, cbase, extra = _worker_chunks()
    sems = (sem0, sem1, sem2)
    _load_idx(sd_hbm, idxb, cbase, extra)

    for b in range(KG_G):
        pltpu.async_copy(t_hbm.at[idxb.at[b, 0]], rows.at[b], sems[b])

    def body(g, carry):
        c0 = g * KG_G
        for b in range(KG_G):
            pltpu.make_async_copy(t_hbm.at[idxb.at[c0 + b, 0]],
                                  rows.at[b], sems[b]).wait()
            pltpu.sync_copy(rows.at[b],
                            out_hbm.at[pl.ds((cbase + c0 + b) * ECH, ECH)])

            @pl.when(g < NGRP_G - 1)
            def _():
                pltpu.async_copy(t_hbm.at[idxb.at[c0 + KG_G + b, 0]],
                                 rows.at[b], sems[b])

        return carry

    lax.fori_loop(0, NGRP_G, body, 0)

    @pl.when(extra)
    def _():
        pltpu.async_copy(t_hbm.at[idxb.at[CPW, 0]], rows.at[0], sem0).wait()
        pltpu.sync_copy(rows.at[0],
                        out_hbm.at[pl.ds((cbase + CPW) * ECH, ECH)])


@functools.partial(
    pl.kernel,
    out_type=jax.ShapeDtypeStruct((NC, NPAD, W128), jnp.float32),
    mesh=_sc_mesh(),
    scratch_types=[
        pltpu.VMEM((CPW + 1, 2, ECH), jnp.int32),
        pltpu.VMEM((KG_S, ECH, W128), jnp.float32),
        pltpu.VMEM_SHARED((NPAD, W128), jnp.float32),
        pltpu.SemaphoreType.DMA,
        pltpu.SemaphoreType.DMA,
    ],
)
def _sc_segsum(vals_hbm, sd_hbm, zeros_hbm, out_hbm, idxb, rows, acc,
               sem0, sem1):
    cid, sid, wid, cbase, extra = _worker_chunks()
    rbase = sid * NROW
    sems = (sem0, sem1)
    _load_idx(sd_hbm, idxb, cbase, extra)
    pltpu.sync_copy(zeros_hbm.at[pl.ds(rbase, NROW)],
                    acc.at[pl.ds(rbase, NROW)])
    plsc.subcore_barrier()

    for b in range(KG_S):
        pltpu.async_copy(vals_hbm.at[pl.ds((cbase + b) * ECH, ECH)],
                         rows.at[b], sems[b])

    def body(g, carry):
        c0 = g * KG_S
        for b in range(KG_S):
            pltpu.make_async_copy(
                vals_hbm.at[pl.ds((cbase + c0 + b) * ECH, ECH)],
                rows.at[b], sems[b]).wait()
            pltpu.sync_copy(rows.at[b], acc.at[idxb.at[c0 + b, 1]], add=True)

            @pl.when(g < NGRP_S - 1)
            def _():
                pltpu.async_copy(
                    vals_hbm.at[pl.ds((cbase + c0 + KG_S + b) * ECH, ECH)],
                    rows.at[b], sems[b])

        return carry

    lax.fori_loop(0, NGRP_S, body, 0)
    c_last = NGRP_S * KG_S
    pltpu.async_copy(vals_hbm.at[pl.ds((cbase + c_last) * ECH, ECH)],
                     rows.at[0], sem0).wait()
    pltpu.sync_copy(rows.at[0], acc.at[idxb.at[c_last, 1]], add=True)

    @pl.when(extra)
    def _():
        pltpu.async_copy(vals_hbm.at[pl.ds((cbase + CPW) * ECH, ECH)],
                         rows.at[0], sem0).wait()
        pltpu.sync_copy(rows.at[0], acc.at[idxb.at[CPW, 1]], add=True)

    plsc.subcore_barrier()
    pltpu.sync_copy(acc.at[pl.ds(rbase, NROW)],
                    out_hbm.at[cid].at[pl.ds(rbase, NROW)])


def kernel(x, edge_index, edge_attr, batch, params):
    sd = jnp.stack([edge_index[0].reshape(ECHUNKS, ECH),
                    edge_index[1].reshape(ECHUNKS, ECH)], axis=1)
    p = params
    row = lambda v: v.reshape(1, -1)

    a2p = [p["A2"][i].reshape(F_IN, H, H).transpose(0, 2, 1).reshape(F_IN, H * H)
           for i in range(3)]
    c2p = [p["c2"][i].reshape(H, H).T.reshape(1, H * H) for i in range(3)]
    sel = (jnp.arange(H * H, dtype=jnp.int32)[:, None] // H
           == jnp.arange(H, dtype=jnp.int32)[None, :]).astype(jnp.float32)
    mw1p = [jnp.pad(p["Mw1"][i], ((0, 0), (0, W128 - H))) for i in range(3)]
    mb1p = [jnp.pad(row(p["Mb1"][i]), ((0, 0), (0, W128 - H))) for i in range(3)]

    zeros_n = jnp.zeros((NPAD, W128), jnp.float32)

    h, t = _tc1(x, p["W0"], row(p["b0"]), mw1p[0], mb1p[0])
    for i in range(3):
        agg = _sc_gather_segsum(t, sd, zeros_n)
        xm, m = _tc2(t, agg, h, p["Mw2"][i], row(p["Mb2"][i]))
        xs = _sc_gather(xm, sd)
        msg = _tc3(edge_attr, xs, p["A1"][i], row(p["c1"][i]), a2p[i], c2p[i], sel)
        agg2 = _sc_segsum(msg, sd, zeros_n)
        if i < 2:
            h, t = _tc4(xm, agg2, p["Wroot"][i], m, mw1p[i + 1], mb1p[i + 1])
        else:
            pooled = _tc4f_pool(xm, agg2, p["Wroot"][i],
                                batch.reshape(N, 1))
    o = _tc5(pooled, p["W1"], row(p["b1"]), p["W2"], row(p["b2"]),
             p["W3"], row(p["b3"]))
    return o.reshape(-1)
